# baseline (speedup 1.0000x reference)
import functools

import jax
import jax.numpy as jnp
from jax import lax
from jax.experimental import pallas as pl
from jax.experimental.pallas import tpu as pltpu
from jax.experimental.pallas import tpu_sc as plsc

F32 = jnp.float32
N_NODES = 50000
N_EDGES = 800000
N_GRAPHS = 64
NPAD = 50176
EPAD = 802816
NB = NPAD // 1024
EB = EPAD // 2048
NC, NS = 2, 16
NW = NC * NS
EPT = EPAD // NW
EPS = EPAD // NS
HPS = EPS // 2
RPT = NPAD // NS


def _lsilu(v, alpha):
    return v * jax.nn.sigmoid(v) + alpha * v


def _node_prep(x_p, batch3, mat_flat, wa, wb, b1):
    def body(x_ref, bt_ref, mat_ref, wa_ref, wb_ref, b1_ref, ab_ref):
        bt = bt_ref[0, 0, :]
        oh = (bt[:, None] == lax.broadcasted_iota(jnp.int32, (1024, N_GRAPHS), 1)
              ).astype(F32)
        M = jnp.dot(oh, mat_ref[...], preferred_element_type=F32)
        xb = x_ref[...]
        cols = [xb[:, 0:1]]
        for j in range(3):
            cols.append(xb[:, 1:2] * M[:, j:j + 1]
                        + xb[:, 2:3] * M[:, 3 + j:4 + j]
                        + xb[:, 3:4] * M[:, 6 + j:7 + j])
        h0 = jnp.concatenate(cols, axis=1)
        a = jnp.dot(h0, wa_ref[...], preferred_element_type=F32) + b1_ref[...]
        b = jnp.dot(h0, wb_ref[...], preferred_element_type=F32)
        ab_ref[...] = jnp.concatenate([a, b], axis=1)

    return pl.pallas_call(
        body,
        grid=(NB,),
        in_specs=[
            pl.BlockSpec((1024, 4), lambda i: (i, 0)),
            pl.BlockSpec((1, 1, 1024), lambda i: (i, 0, 0)),
            pl.BlockSpec((N_GRAPHS, 9), lambda i: (0, 0)),
            pl.BlockSpec((4, 64), lambda i: (0, 0)),
            pl.BlockSpec((4, 64), lambda i: (0, 0)),
            pl.BlockSpec((1, 64), lambda i: (0, 0)),
        ],
        out_specs=pl.BlockSpec((1024, 128), lambda i: (i, 0)),
        out_shape=jax.ShapeDtypeStruct((NPAD, 128), F32),
    )(x_p, batch3, mat_flat, wa, wb, b1)


def _node_update(agg, iw1, ib1, iw2, ib2, wa, wb, nb1):
    d = agg.shape[1]

    def body(g_ref, iw1_ref, ib1_ref, iw2_ref, ib2_ref, wa_ref, wb_ref,
             nb1_ref, ab_ref):
        h = _lsilu(g_ref[...], 0.1)
        hi = _lsilu(jnp.dot(h, iw1_ref[...], preferred_element_type=F32)
                    + ib1_ref[...], 0.05)
        hi = _lsilu(jnp.dot(hi, iw2_ref[...], preferred_element_type=F32)
                    + ib2_ref[...], 0.05)
        h = hi + h
        a = jnp.dot(h, wa_ref[...], preferred_element_type=F32) + nb1_ref[...]
        b = jnp.dot(h, wb_ref[...], preferred_element_type=F32)
        ab_ref[...] = jnp.concatenate([a, b], axis=1)

    return pl.pallas_call(
        body,
        grid=(NB,),
        in_specs=[
            pl.BlockSpec((1024, d), lambda i: (i, 0)),
            pl.BlockSpec(iw1.shape, lambda i: (0, 0)),
            pl.BlockSpec(ib1.shape, lambda i: (0, 0)),
            pl.BlockSpec(iw2.shape, lambda i: (0, 0)),
            pl.BlockSpec(ib2.shape, lambda i: (0, 0)),
            pl.BlockSpec((d, 64), lambda i: (0, 0)),
            pl.BlockSpec((d, 64), lambda i: (0, 0)),
            pl.BlockSpec((1, 64), lambda i: (0, 0)),
        ],
        out_specs=pl.BlockSpec((1024, 128), lambda i: (i, 0)),
        out_shape=jax.ShapeDtypeStruct((NPAD, 128), F32),
    )(agg, iw1, ib1, iw2, ib2, wa, wb, nb1)


def _edge_mm_packed(zp, w2, b2):
    dout = w2.shape[1]
    wbd = jnp.zeros((128, 128), F32)
    wbd = wbd.at[0:64, 0:dout].set(w2)
    wbd = wbd.at[64:128, 64:64 + dout].set(w2)
    bbd = jnp.zeros((1, 128), F32)
    bbd = bbd.at[:, 0:dout].set(b2)
    bbd = bbd.at[:, 64:64 + dout].set(b2)

    def body(z_ref, w_ref, b_ref, y_ref):
        za = _lsilu(z_ref[...], 0.05)
        yy = jnp.dot(za, w_ref[...], preferred_element_type=F32) + b_ref[...]
        y_ref[...] = _lsilu(yy, 0.05)

    return pl.pallas_call(
        body,
        grid=(EB,),
        in_specs=[
            pl.BlockSpec((1024, 128), lambda i: (i, 0)),
            pl.BlockSpec((128, 128), lambda i: (0, 0)),
            pl.BlockSpec((1, 128), lambda i: (0, 0)),
        ],
        out_specs=pl.BlockSpec((1024, 128), lambda i: (i, 0)),
        out_shape=jax.ShapeDtypeStruct((EPAD // 2, 128), F32),
    )(zp, wbd, bbd)


def _edge_mm_split(zp, w2, b2):
    def body(z_ref, w_ref, b_ref, ye_ref, yo_ref):
        za = _lsilu(z_ref[...], 0.05)
        w = w_ref[...]
        b = b_ref[...]
        ye_ref[...] = _lsilu(
            jnp.dot(za[:, 0:64], w, preferred_element_type=F32) + b, 0.05)
        yo_ref[...] = _lsilu(
            jnp.dot(za[:, 64:128], w, preferred_element_type=F32) + b, 0.05)

    return pl.pallas_call(
        body,
        grid=(EB,),
        in_specs=[
            pl.BlockSpec((1024, 128), lambda i: (i, 0)),
            pl.BlockSpec((64, 128), lambda i: (0, 0)),
            pl.BlockSpec((1, 128), lambda i: (0, 0)),
        ],
        out_specs=[pl.BlockSpec((1024, 128), lambda i: (i, 0))] * 2,
        out_shape=[jax.ShapeDtypeStruct((EPAD // 2, 128), F32)] * 2,
    )(zp, w2, b2)


def _final(agg, batch3, fcw, fcb):
    def body(g_ref, bt_ref, w_ref, b_ref, o_ref, acc_ref):
        i = pl.program_id(0)
        h = _lsilu(g_ref[...], 0.1)
        bt = bt_ref[0, 0, :]
        oh = (bt[:, None] == lax.broadcasted_iota(jnp.int32, (1024, N_GRAPHS), 1)
              ).astype(F32)
        part = lax.dot_general(oh, h, (((0,), (0,)), ((), ())),
                               preferred_element_type=F32)

        @pl.when(i == 0)
        def _():
            acc_ref[...] = part

        @pl.when(i > 0)
        def _():
            acc_ref[...] = acc_ref[...] + part

        @pl.when(i == NB - 1)
        def _():
            o = jnp.dot(acc_ref[...], w_ref[...], preferred_element_type=F32) \
                + b_ref[...]
            o = -_lsilu(o, 0.1) * 0.1
            o_ref[...] = jnp.broadcast_to(o, (N_GRAPHS, 128))

    return pl.pallas_call(
        body,
        grid=(NB,),
        in_specs=[
            pl.BlockSpec((1024, 128), lambda i: (i, 0)),
            pl.BlockSpec((1, 1, 1024), lambda i: (i, 0, 0)),
            pl.BlockSpec((128, 1), lambda i: (0, 0)),
            pl.BlockSpec((1, 1), lambda i: (0, 0)),
        ],
        out_specs=pl.BlockSpec((N_GRAPHS, 128), lambda i: (0, 0)),
        out_shape=jax.ShapeDtypeStruct((N_GRAPHS, 128), F32),
        scratch_shapes=[pltpu.VMEM((N_GRAPHS, 128), F32)],
    )(agg, batch3, fcw, fcb)


def _sc_gather(tab, dst_g, src_g):
    mesh = plsc.VectorSubcoreMesh(core_axis_name="c", subcore_axis_name="s")
    MC = 256
    NMC = EPT // MC
    PR = MC // 2

    @functools.partial(
        pl.kernel,
        out_type=jax.ShapeDtypeStruct((EPAD // 2, 128), F32),
        mesh=mesh,
        compiler_params=pltpu.CompilerParams(use_tc_tiling_on_sc=False),
        scratch_types=[
            pltpu.VMEM((2, 128), jnp.int32),
            pltpu.VMEM((2, 128), jnp.int32),
            pltpu.VMEM((2, 128), jnp.int32),
            pltpu.VMEM((2, 128), jnp.int32),
            pltpu.VMEM((MC, 64), F32),
            pltpu.VMEM((MC, 64), F32),
            pltpu.VMEM((MC, 64), F32),
            pltpu.VMEM((MC, 64), F32),
            pltpu.VMEM((PR, 128), F32),
            pltpu.VMEM((PR, 128), F32),
            pltpu.SemaphoreType.DMA,
            pltpu.SemaphoreType.DMA,
            pltpu.SemaphoreType.DMA,
            pltpu.SemaphoreType.DMA,
            pltpu.SemaphoreType.DMA,
            pltpu.SemaphoreType.DMA,
        ],
    )
    def k(t_hbm, d_hbm, s_hbm, z_hbm, idd0, idd1, ids0, ids1,
          ra0, ra1, rb0, rb1, zp0, zp1, six0, six1, sg0, sg1, st0, st1):
        idd = (idd0, idd1)
        ids_ = (ids0, ids1)
        ra = (ra0, ra1)
        rb = (rb0, rb1)
        zp = (zp0, zp1)
        six = (six0, six1)
        sg = (sg0, sg1)
        st = (st0, st1)
        wid = lax.axis_index("s") * NC + lax.axis_index("c")
        ebase = wid * EPT
        rbase = wid * (EPT // 128)

        def fire_idx(m, b):
            r0 = rbase + m * 2
            pltpu.async_copy(d_hbm.at[pl.ds(r0, 2)], idd[b], six[b])
            pltpu.async_copy(s_hbm.at[pl.ds(r0, 2)], ids_[b], six[b])

        def wait_idx(b):
            pltpu.make_async_copy(d_hbm.at[pl.ds(0, 2)], idd[b], six[b]).wait()
            pltpu.make_async_copy(s_hbm.at[pl.ds(0, 2)], ids_[b], six[b]).wait()

        def fire_gathers(b):
            for j in range(2):
                pltpu.async_copy(
                    t_hbm.at[idd[b].at[j]], ra[b].at[pl.ds(j * 128, 128)], sg[b])
                pltpu.async_copy(
                    t_hbm.at[ids_[b].at[j]], rb[b].at[pl.ds(j * 128, 128)], sg[b])

        def wait_gathers(b):
            pltpu.make_async_copy(t_hbm.at[pl.ds(0, MC)], ra[b], sg[b]).wait()
            pltpu.make_async_copy(t_hbm.at[pl.ds(0, MC)], rb[b], sg[b]).wait()

        def drain_store(b):
            pltpu.make_async_copy(zp[b], z_hbm.at[pl.ds(0, PR)], st[b]).wait()

        fire_idx(0, 0)
        fire_idx(1, 1)
        wait_idx(0)
        fire_gathers(0)

        def body(m2, carry):
            for b in range(2):
                m = 2 * m2 + b
                wait_gathers(b)

                @pl.when(m >= 2)
                def _(b=b):
                    drain_store(b)

                def addrow(q, c2, b=b):
                    for e in range(2):
                        for kk in range(4):
                            sl = pl.ds(kk * 16, 16)
                            zp[b][q, pl.ds(e * 64 + kk * 16, 16)] = (
                                ra[b][2 * q + e, sl] + rb[b][2 * q + e, sl])
                    return c2

                lax.fori_loop(0, PR, addrow, 0, unroll=4)
                off = ebase + m * MC
                pltpu.async_copy(zp[b], z_hbm.at[pl.ds(off // 2, PR)], st[b])
                bn = 1 - b

                @pl.when(m + 1 <= NMC - 1)
                def _(b=b, bn=bn):
                    wait_idx(bn)
                    fire_gathers(bn)

                @pl.when(m + 2 <= NMC - 1)
                def _(m=m, b=b):
                    fire_idx(m + 2, b)

            return carry

        lax.fori_loop(0, NMC // 2, body, 0)
        drain_store(0)
        drain_store(1)

    return k(tab, dst_g, src_g)


def _sc_scatter(ya, yb, cola, colb, dst_e2, dst_o2, nch):
    mesh = plsc.VectorSubcoreMesh(core_axis_name="c", subcore_axis_name="s")
    MC = 896
    NMC = HPS // MC
    NSUB = MC // 128
    IR = HPS // 128
    nhalf = nch // 2
    dreal = nch * 16

    @functools.partial(
        pl.kernel,
        out_type=jax.ShapeDtypeStruct((NPAD, dreal), F32),
        mesh=mesh,
        compiler_params=pltpu.CompilerParams(use_tc_tiling_on_sc=False),
        scratch_types=[
            pltpu.VMEM((NSUB, 128), jnp.int32),
            pltpu.VMEM((NSUB, 128), jnp.int32),
            pltpu.VMEM((MC, 16), F32),
            pltpu.VMEM((MC, 16), F32),
            pltpu.VMEM((128,), jnp.int32),
            pltpu.VMEM((128,), jnp.int32),
            pltpu.VMEM((128,), jnp.int32),
            pltpu.VMEM((128,), jnp.int32),
            pltpu.VMEM((128,), jnp.int32),
            pltpu.VMEM((128,), jnp.int32),
            pltpu.VMEM((128,), jnp.int32),
            pltpu.VMEM((392, 16), F32),
            pltpu.VMEM((392, 16), F32),
            pltpu.VMEM_SHARED((NPAD, 16), F32),
            pltpu.SemaphoreType.DMA,
            pltpu.SemaphoreType.DMA,
            pltpu.SemaphoreType.DMA,
            pltpu.SemaphoreType.DMA,
            pltpu.SemaphoreType.DMA,
        ],
    )
    def k(ya_hbm, yb_hbm, de_hbm, do_hbm, agg_hbm, idx0, idx1, yv0, yv1,
          q0, q1, q2, q3, q4, q5, q6, zb, bb, accum, sy0, sy1, ssc, six0,
          six1):
        yv = (yv0, yv1)
        sy = (sy0, sy1)
        six = (six0, six1)
        idxc = (idx0, idx1)
        idsb = (q0, q1, q2, q3, q4, q5, q6)
        core = lax.axis_index("c")
        sid = lax.axis_index("s")
        ebase = sid * HPS
        nbase = sid * RPT
        rbase = sid * IR

        def zrow(r, c2):
            zb[r, pl.ds(0, 16)] = jnp.zeros((16,), F32)
            return c2

        lax.fori_loop(0, 392, zrow, 0)

        for ci in range(nhalf):
            cc = 2 * ci + core
            for t in range(8):
                pltpu.async_copy(
                    zb, accum.at[pl.ds(nbase + t * 392, 392)], sy0)
            for t in range(8):
                pltpu.make_async_copy(
                    zb, accum.at[pl.ds(nbase + t * 392, 392)], sy0).wait()
            plsc.subcore_barrier()

            for y_hbm, d_hbm, colbase in ((ya_hbm, de_hbm, cola),
                                          (yb_hbm, do_hbm, colb)):
                col = colbase + cc * 16

                def fire_load(m, b, y_hbm=y_hbm, d_hbm=d_hbm, col=col):
                    off = ebase + m * MC
                    pltpu.async_copy(
                        y_hbm.at[pl.ds(off, MC), pl.ds(col, 16)], yv[b], sy[b])
                    pltpu.async_copy(
                        d_hbm.at[pl.ds(rbase + m * NSUB, NSUB)], idxc[b],
                        six[b])

                def wait_load(b, y_hbm=y_hbm, d_hbm=d_hbm, col=col):
                    pltpu.make_async_copy(
                        y_hbm.at[pl.ds(0, MC), pl.ds(col, 16)], yv[b],
                        sy[b]).wait()
                    pltpu.make_async_copy(
                        d_hbm.at[pl.ds(0, NSUB)], idxc[b], six[b]).wait()

                def drain_sc(b):
                    for q in range(NSUB):
                        pltpu.make_async_copy(
                            yv[b].at[pl.ds(q * 128, 128)],
                            accum.at[idsb[q]], ssc).wait()

                fire_load(0, 0)
                fire_load(1, 1)

                def body(m2, carry, fire_load=fire_load,
                         wait_load=wait_load, drain_sc=drain_sc):
                    for b in range(2):
                        m = 2 * m2 + b
                        wait_load(b)
                        for q in range(NSUB):
                            for kk in range(8):
                                sl = pl.ds(kk * 16, 16)
                                idsb[q][sl] = idxc[b][q, sl]
                            pltpu.async_copy(
                                yv[b].at[pl.ds(q * 128, 128)],
                                accum.at[idsb[q]], ssc, add=True)
                        drain_sc(b)

                        @pl.when(m + 2 <= NMC - 1)
                        def _(m=m, b=b):
                            fire_load(m + 2, b)

                    return carry

                lax.fori_loop(0, NMC // 2, body, 0)

            plsc.subcore_barrier()
            for t in range(8):
                r = nbase + t * 392
                pltpu.async_copy(
                    accum.at[pl.ds(r, 392)],
                    agg_hbm.at[pl.ds(r, 392), pl.ds(cc * 16, 16)], sy1)
            for t in range(8):
                r = nbase + t * 392
                pltpu.make_async_copy(
                    accum.at[pl.ds(r, 392)],
                    agg_hbm.at[pl.ds(r, 392), pl.ds(cc * 16, 16)], sy1).wait()
            plsc.subcore_barrier()

    return k(ya, yb, dst_e2, dst_o2)


def _split_conv(conv):
    w1 = conv['l1']['w']
    d = w1.shape[0] // 2
    return (w1[:d], w1[d:], conv['l1']['b'][None, :],
            conv['l2']['w'], conv['l2']['b'][None, :])


def kernel(x, matrix, batch, edge_index, params):
    p = params
    x_p = jnp.pad(x.astype(F32), ((0, NPAD - N_NODES), (0, 0)))
    batch_p = jnp.concatenate(
        [batch.astype(jnp.int32),
         jnp.full((NPAD - N_NODES,), N_GRAPHS, jnp.int32)])
    batch3 = batch_p.reshape(NB, 1, 1024)
    ei = edge_index.astype(jnp.int32)
    pad_e = jnp.full((EPAD - N_EDGES,), N_NODES, jnp.int32)
    src = jnp.concatenate([ei[0], pad_e])
    dst = jnp.concatenate([ei[1], pad_e])
    dst_g = (2 * dst).reshape(-1, 128)
    src_g = (2 * src + 1).reshape(-1, 128)
    dst_e2 = dst[0::2].reshape(-1, 128)
    dst_o2 = dst[1::2].reshape(-1, 128)
    mat_flat = matrix.astype(F32).reshape(N_GRAPHS, 9)

    def edge_stage(ab, w2, b2):
        tab = ab.reshape(2 * NPAD, 64)
        zp = _sc_gather(tab, dst_g, src_g)
        dout = w2.shape[1]
        if dout == 128:
            ye, yo = _edge_mm_split(zp, w2, b2)
            return _sc_scatter(ye, yo, 0, 0, dst_e2, dst_o2, dout // 16)
        yp = _edge_mm_packed(zp, w2, b2)
        return _sc_scatter(yp, yp, 0, 64, dst_e2, dst_o2, dout // 16)

    wa, wb, b1, w2, b2 = _split_conv(p['conv1'])
    ab = _node_prep(x_p, batch3, mat_flat, wa, wb, b1)
    agg = edge_stage(ab, w2, b2)

    for conv_name, inter_name in (('conv2', 'inter1'),
                                  ('conv3', 'inter2'),
                                  ('conv4', 'inter3')):
        wa, wb, b1, w2, b2 = _split_conv(p[conv_name])
        it = p[inter_name]
        ab = _node_update(
            agg, it['l1']['w'], it['l1']['b'][None, :],
            it['l2']['w'], it['l2']['b'][None, :], wa, wb, b1)
        agg = edge_stage(ab, w2, b2)

    out128 = _final(agg, batch3, p['fc1']['w'], p['fc1']['b'][None, :])
    return out128[:, :1]


# speedup vs baseline: 1.1497x; 1.1497x over previous
import functools

import jax
import jax.numpy as jnp
from jax import lax
from jax.experimental import pallas as pl
from jax.experimental.pallas import tpu as pltpu
from jax.experimental.pallas import tpu_sc as plsc

F32 = jnp.float32
BF16 = jnp.bfloat16
N_NODES = 50000
N_EDGES = 800000
N_GRAPHS = 64
NPAD = 50176
EPAD = 802816
NB = NPAD // 1024
EB = EPAD // 2048
NC, NS = 2, 16
NW = NC * NS
EPT = EPAD // NW
EPS = EPAD // NS
HPS = EPS // 2
RPT = NPAD // NS


def _lsilu(v, alpha):
    return v * jax.nn.sigmoid(v) + alpha * v


def _node_prep(x_p, batch3, mat_flat, wa, wb, b1):
    def body(x_ref, bt_ref, mat_ref, wa_ref, wb_ref, b1_ref, ab_ref):
        bt = bt_ref[0, 0, :]
        oh = (bt[:, None] == lax.broadcasted_iota(jnp.int32, (1024, N_GRAPHS), 1)
              ).astype(F32)
        M = jnp.dot(oh, mat_ref[...], preferred_element_type=F32)
        xb = x_ref[...]
        cols = [xb[:, 0:1]]
        for j in range(3):
            cols.append(xb[:, 1:2] * M[:, j:j + 1]
                        + xb[:, 2:3] * M[:, 3 + j:4 + j]
                        + xb[:, 3:4] * M[:, 6 + j:7 + j])
        h0 = jnp.concatenate(cols, axis=1)
        a = jnp.dot(h0, wa_ref[...], preferred_element_type=F32) + b1_ref[...]
        b = jnp.dot(h0, wb_ref[...], preferred_element_type=F32)
        ab_ref[...] = jnp.concatenate([a, b], axis=1).astype(BF16)

    return pl.pallas_call(
        body,
        grid=(NB,),
        in_specs=[
            pl.BlockSpec((1024, 4), lambda i: (i, 0)),
            pl.BlockSpec((1, 1, 1024), lambda i: (i, 0, 0)),
            pl.BlockSpec((N_GRAPHS, 9), lambda i: (0, 0)),
            pl.BlockSpec((4, 64), lambda i: (0, 0)),
            pl.BlockSpec((4, 64), lambda i: (0, 0)),
            pl.BlockSpec((1, 64), lambda i: (0, 0)),
        ],
        out_specs=pl.BlockSpec((1024, 128), lambda i: (i, 0)),
        out_shape=jax.ShapeDtypeStruct((NPAD, 128), BF16),
    )(x_p, batch3, mat_flat, wa, wb, b1)


def _node_update(agg, iw1, ib1, iw2, ib2, wa, wb, nb1):
    d = agg.shape[1]

    def body(g_ref, iw1_ref, ib1_ref, iw2_ref, ib2_ref, wa_ref, wb_ref,
             nb1_ref, ab_ref):
        h = _lsilu(g_ref[...], 0.1)
        hi = _lsilu(jnp.dot(h, iw1_ref[...], preferred_element_type=F32)
                    + ib1_ref[...], 0.05)
        hi = _lsilu(jnp.dot(hi, iw2_ref[...], preferred_element_type=F32)
                    + ib2_ref[...], 0.05)
        h = hi + h
        a = jnp.dot(h, wa_ref[...], preferred_element_type=F32) + nb1_ref[...]
        b = jnp.dot(h, wb_ref[...], preferred_element_type=F32)
        ab_ref[...] = jnp.concatenate([a, b], axis=1).astype(BF16)

    return pl.pallas_call(
        body,
        grid=(NB,),
        in_specs=[
            pl.BlockSpec((1024, d), lambda i: (i, 0)),
            pl.BlockSpec(iw1.shape, lambda i: (0, 0)),
            pl.BlockSpec(ib1.shape, lambda i: (0, 0)),
            pl.BlockSpec(iw2.shape, lambda i: (0, 0)),
            pl.BlockSpec(ib2.shape, lambda i: (0, 0)),
            pl.BlockSpec((d, 64), lambda i: (0, 0)),
            pl.BlockSpec((d, 64), lambda i: (0, 0)),
            pl.BlockSpec((1, 64), lambda i: (0, 0)),
        ],
        out_specs=pl.BlockSpec((1024, 128), lambda i: (i, 0)),
        out_shape=jax.ShapeDtypeStruct((NPAD, 128), BF16),
    )(agg, iw1, ib1, iw2, ib2, wa, wb, nb1)


def _edge_mm_packed(zp, w2, b2):
    dout = w2.shape[1]
    wbd = jnp.zeros((128, 128), F32)
    wbd = wbd.at[0:64, 0:dout].set(w2)
    wbd = wbd.at[64:128, 64:64 + dout].set(w2)
    bbd = jnp.zeros((1, 128), F32)
    bbd = bbd.at[:, 0:dout].set(b2)
    bbd = bbd.at[:, 64:64 + dout].set(b2)

    def body(z_ref, w_ref, b_ref, y_ref):
        za = _lsilu(z_ref[...].astype(F32), 0.05)
        yy = jnp.dot(za, w_ref[...], preferred_element_type=F32) + b_ref[...]
        y_ref[...] = _lsilu(yy, 0.05)

    return pl.pallas_call(
        body,
        grid=(EB,),
        in_specs=[
            pl.BlockSpec((1024, 128), lambda i: (i, 0)),
            pl.BlockSpec((128, 128), lambda i: (0, 0)),
            pl.BlockSpec((1, 128), lambda i: (0, 0)),
        ],
        out_specs=pl.BlockSpec((1024, 128), lambda i: (i, 0)),
        out_shape=jax.ShapeDtypeStruct((EPAD // 2, 128), F32),
    )(zp, wbd, bbd)


def _edge_mm_split(zp, w2, b2):
    def body(z_ref, w_ref, b_ref, ye_ref, yo_ref):
        za = _lsilu(z_ref[...].astype(F32), 0.05)
        w = w_ref[...]
        b = b_ref[...]
        ye_ref[...] = _lsilu(
            jnp.dot(za[:, 0:64], w, preferred_element_type=F32) + b, 0.05)
        yo_ref[...] = _lsilu(
            jnp.dot(za[:, 64:128], w, preferred_element_type=F32) + b, 0.05)

    return pl.pallas_call(
        body,
        grid=(EB,),
        in_specs=[
            pl.BlockSpec((1024, 128), lambda i: (i, 0)),
            pl.BlockSpec((64, 128), lambda i: (0, 0)),
            pl.BlockSpec((1, 128), lambda i: (0, 0)),
        ],
        out_specs=[pl.BlockSpec((1024, 128), lambda i: (i, 0))] * 2,
        out_shape=[jax.ShapeDtypeStruct((EPAD // 2, 128), F32)] * 2,
    )(zp, w2, b2)


def _final(agg, batch3, fcw, fcb):
    def body(g_ref, bt_ref, w_ref, b_ref, o_ref, acc_ref):
        i = pl.program_id(0)
        h = _lsilu(g_ref[...], 0.1)
        bt = bt_ref[0, 0, :]
        oh = (bt[:, None] == lax.broadcasted_iota(jnp.int32, (1024, N_GRAPHS), 1)
              ).astype(F32)
        part = lax.dot_general(oh, h, (((0,), (0,)), ((), ())),
                               preferred_element_type=F32)

        @pl.when(i == 0)
        def _():
            acc_ref[...] = part

        @pl.when(i > 0)
        def _():
            acc_ref[...] = acc_ref[...] + part

        @pl.when(i == NB - 1)
        def _():
            o = jnp.dot(acc_ref[...], w_ref[...], preferred_element_type=F32) \
                + b_ref[...]
            o = -_lsilu(o, 0.1) * 0.1
            o_ref[...] = jnp.broadcast_to(o, (N_GRAPHS, 128))

    return pl.pallas_call(
        body,
        grid=(NB,),
        in_specs=[
            pl.BlockSpec((1024, 128), lambda i: (i, 0)),
            pl.BlockSpec((1, 1, 1024), lambda i: (i, 0, 0)),
            pl.BlockSpec((128, 1), lambda i: (0, 0)),
            pl.BlockSpec((1, 1), lambda i: (0, 0)),
        ],
        out_specs=pl.BlockSpec((N_GRAPHS, 128), lambda i: (0, 0)),
        out_shape=jax.ShapeDtypeStruct((N_GRAPHS, 128), F32),
        scratch_shapes=[pltpu.VMEM((N_GRAPHS, 128), F32)],
    )(agg, batch3, fcw, fcb)


def _sc_gather(tab, dst_g, src_g):
    mesh = plsc.VectorSubcoreMesh(core_axis_name="c", subcore_axis_name="s")
    MC = 256
    NMC = EPT // MC
    PR = MC // 2

    @functools.partial(
        pl.kernel,
        out_type=jax.ShapeDtypeStruct((EPAD // 2, 128), BF16),
        mesh=mesh,
        compiler_params=pltpu.CompilerParams(use_tc_tiling_on_sc=False),
        scratch_types=[
            pltpu.VMEM((2, 128), jnp.int32),
            pltpu.VMEM((2, 128), jnp.int32),
            pltpu.VMEM((2, 128), jnp.int32),
            pltpu.VMEM((2, 128), jnp.int32),
            pltpu.VMEM((MC, 64), BF16),
            pltpu.VMEM((MC, 64), BF16),
            pltpu.VMEM((MC, 64), BF16),
            pltpu.VMEM((MC, 64), BF16),
            pltpu.VMEM((PR, 128), BF16),
            pltpu.VMEM((PR, 128), BF16),
            pltpu.SemaphoreType.DMA,
            pltpu.SemaphoreType.DMA,
            pltpu.SemaphoreType.DMA,
            pltpu.SemaphoreType.DMA,
            pltpu.SemaphoreType.DMA,
            pltpu.SemaphoreType.DMA,
        ],
    )
    def k(t_hbm, d_hbm, s_hbm, z_hbm, idd0, idd1, ids0, ids1,
          ra0, ra1, rb0, rb1, zp0, zp1, six0, six1, sg0, sg1, st0, st1):
        idd = (idd0, idd1)
        ids_ = (ids0, ids1)
        ra = (ra0, ra1)
        rb = (rb0, rb1)
        zp = (zp0, zp1)
        six = (six0, six1)
        sg = (sg0, sg1)
        st = (st0, st1)
        wid = lax.axis_index("s") * NC + lax.axis_index("c")
        ebase = wid * EPT
        rbase = wid * (EPT // 128)

        def fire_idx(m, b):
            r0 = rbase + m * 2
            pltpu.async_copy(d_hbm.at[pl.ds(r0, 2)], idd[b], six[b])
            pltpu.async_copy(s_hbm.at[pl.ds(r0, 2)], ids_[b], six[b])

        def wait_idx(b):
            pltpu.make_async_copy(d_hbm.at[pl.ds(0, 2)], idd[b], six[b]).wait()
            pltpu.make_async_copy(s_hbm.at[pl.ds(0, 2)], ids_[b], six[b]).wait()

        def fire_gathers(b):
            for j in range(2):
                pltpu.async_copy(
                    t_hbm.at[idd[b].at[j]], ra[b].at[pl.ds(j * 128, 128)], sg[b])
                pltpu.async_copy(
                    t_hbm.at[ids_[b].at[j]], rb[b].at[pl.ds(j * 128, 128)], sg[b])

        def wait_gathers(b):
            pltpu.make_async_copy(t_hbm.at[pl.ds(0, MC)], ra[b], sg[b]).wait()
            pltpu.make_async_copy(t_hbm.at[pl.ds(0, MC)], rb[b], sg[b]).wait()

        def drain_store(b):
            pltpu.make_async_copy(zp[b], z_hbm.at[pl.ds(0, PR)], st[b]).wait()

        fire_idx(0, 0)
        fire_idx(1, 1)
        wait_idx(0)
        fire_gathers(0)

        def body(m2, carry):
            for b in range(2):
                m = 2 * m2 + b
                wait_gathers(b)

                @pl.when(m >= 2)
                def _(b=b):
                    drain_store(b)

                def addrow(q, c2, b=b):
                    for e in range(2):
                        for kk in range(2):
                            sl = pl.ds(kk * 32, 32)
                            zp[b][q, pl.ds(e * 64 + kk * 32, 32)] = (
                                ra[b][2 * q + e, sl] + rb[b][2 * q + e, sl])
                    return c2

                lax.fori_loop(0, PR, addrow, 0)
                off = ebase + m * MC
                pltpu.async_copy(zp[b], z_hbm.at[pl.ds(off // 2, PR)], st[b])
                bn = 1 - b

                @pl.when(m + 1 <= NMC - 1)
                def _(b=b, bn=bn):
                    wait_idx(bn)
                    fire_gathers(bn)

                @pl.when(m + 2 <= NMC - 1)
                def _(m=m, b=b):
                    fire_idx(m + 2, b)

            return carry

        lax.fori_loop(0, NMC // 2, body, 0)
        drain_store(0)
        drain_store(1)

    return k(tab, dst_g, src_g)


def _sc_scatter(ya, yb, cola, colb, dst_e2, dst_o2, nch):
    mesh = plsc.VectorSubcoreMesh(core_axis_name="c", subcore_axis_name="s")
    MC = 896
    NMC = HPS // MC
    NSUB = MC // 128
    IR = HPS // 128
    nhalf = nch // 2
    dreal = nch * 16

    @functools.partial(
        pl.kernel,
        out_type=jax.ShapeDtypeStruct((NPAD, dreal), F32),
        mesh=mesh,
        compiler_params=pltpu.CompilerParams(use_tc_tiling_on_sc=False),
        scratch_types=[
            pltpu.VMEM((NSUB, 128), jnp.int32),
            pltpu.VMEM((NSUB, 128), jnp.int32),
            pltpu.VMEM((MC, 16), F32),
            pltpu.VMEM((MC, 16), F32),
            pltpu.VMEM((128,), jnp.int32),
            pltpu.VMEM((128,), jnp.int32),
            pltpu.VMEM((128,), jnp.int32),
            pltpu.VMEM((128,), jnp.int32),
            pltpu.VMEM((128,), jnp.int32),
            pltpu.VMEM((128,), jnp.int32),
            pltpu.VMEM((128,), jnp.int32),
            pltpu.VMEM((392, 16), F32),
            pltpu.VMEM((392, 16), F32),
            pltpu.VMEM_SHARED((NPAD, 16), F32),
            pltpu.SemaphoreType.DMA,
            pltpu.SemaphoreType.DMA,
            pltpu.SemaphoreType.DMA,
            pltpu.SemaphoreType.DMA,
            pltpu.SemaphoreType.DMA,
        ],
    )
    def k(ya_hbm, yb_hbm, de_hbm, do_hbm, agg_hbm, idx0, idx1, yv0, yv1,
          q0, q1, q2, q3, q4, q5, q6, zb, bb, accum, sy0, sy1, ssc, six0,
          six1):
        yv = (yv0, yv1)
        sy = (sy0, sy1)
        six = (six0, six1)
        idxc = (idx0, idx1)
        idsb = (q0, q1, q2, q3, q4, q5, q6)
        core = lax.axis_index("c")
        sid = lax.axis_index("s")
        ebase = sid * HPS
        nbase = sid * RPT
        rbase = sid * IR

        def zrow(r, c2):
            zb[r, pl.ds(0, 16)] = jnp.zeros((16,), F32)
            return c2

        lax.fori_loop(0, 392, zrow, 0)

        for ci in range(nhalf):
            cc = 2 * ci + core
            for t in range(8):
                pltpu.sync_copy(zb, accum.at[pl.ds(nbase + t * 392, 392)])
            plsc.subcore_barrier()

            for y_hbm, d_hbm, colbase in ((ya_hbm, de_hbm, cola),
                                          (yb_hbm, do_hbm, colb)):
                col = colbase + cc * 16

                def fire_load(m, b, y_hbm=y_hbm, d_hbm=d_hbm, col=col):
                    off = ebase + m * MC
                    pltpu.async_copy(
                        y_hbm.at[pl.ds(off, MC), pl.ds(col, 16)], yv[b], sy[b])
                    pltpu.async_copy(
                        d_hbm.at[pl.ds(rbase + m * NSUB, NSUB)], idxc[b],
                        six[b])

                def wait_load(b, y_hbm=y_hbm, d_hbm=d_hbm, col=col):
                    pltpu.make_async_copy(
                        y_hbm.at[pl.ds(0, MC), pl.ds(col, 16)], yv[b],
                        sy[b]).wait()
                    pltpu.make_async_copy(
                        d_hbm.at[pl.ds(0, NSUB)], idxc[b], six[b]).wait()

                def drain_sc(b):
                    for q in range(NSUB):
                        pltpu.make_async_copy(
                            yv[b].at[pl.ds(q * 128, 128)],
                            accum.at[idsb[q]], ssc).wait()

                fire_load(0, 0)
                fire_load(1, 1)

                def body(m2, carry, fire_load=fire_load,
                         wait_load=wait_load, drain_sc=drain_sc):
                    for b in range(2):
                        m = 2 * m2 + b
                        wait_load(b)
                        for q in range(NSUB):
                            for kk in range(8):
                                sl = pl.ds(kk * 16, 16)
                                idsb[q][sl] = idxc[b][q, sl]
                            pltpu.async_copy(
                                yv[b].at[pl.ds(q * 128, 128)],
                                accum.at[idsb[q]], ssc, add=True)
                        drain_sc(b)

                        @pl.when(m + 2 <= NMC - 1)
                        def _(m=m, b=b):
                            fire_load(m + 2, b)

                    return carry

                lax.fori_loop(0, NMC // 2, body, 0)

            plsc.subcore_barrier()
            for t in range(8):
                r = nbase + t * 392
                pltpu.sync_copy(accum.at[pl.ds(r, 392)], bb)
                pltpu.sync_copy(
                    bb, agg_hbm.at[pl.ds(r, 392), pl.ds(cc * 16, 16)])
            plsc.subcore_barrier()

    return k(ya, yb, dst_e2, dst_o2)


def _split_conv(conv):
    w1 = conv['l1']['w']
    d = w1.shape[0] // 2
    return (w1[:d], w1[d:], conv['l1']['b'][None, :],
            conv['l2']['w'], conv['l2']['b'][None, :])


def kernel(x, matrix, batch, edge_index, params):
    p = params
    x_p = jnp.pad(x.astype(F32), ((0, NPAD - N_NODES), (0, 0)))
    batch_p = jnp.concatenate(
        [batch.astype(jnp.int32),
         jnp.full((NPAD - N_NODES,), N_GRAPHS, jnp.int32)])
    batch3 = batch_p.reshape(NB, 1, 1024)
    ei = edge_index.astype(jnp.int32)
    pad_e = jnp.full((EPAD - N_EDGES,), N_NODES, jnp.int32)
    src = jnp.concatenate([ei[0], pad_e])
    dst = jnp.concatenate([ei[1], pad_e])
    dst_g = (2 * dst).reshape(-1, 128)
    src_g = (2 * src + 1).reshape(-1, 128)
    dst_e2 = dst[0::2].reshape(-1, 128)
    dst_o2 = dst[1::2].reshape(-1, 128)
    mat_flat = matrix.astype(F32).reshape(N_GRAPHS, 9)

    def edge_stage(ab, w2, b2):
        tab = ab.reshape(2 * NPAD, 64)
        zp = _sc_gather(tab, dst_g, src_g)
        dout = w2.shape[1]
        if dout == 128:
            ye, yo = _edge_mm_split(zp, w2, b2)
            return _sc_scatter(ye, yo, 0, 0, dst_e2, dst_o2, dout // 16)
        yp = _edge_mm_packed(zp, w2, b2)
        return _sc_scatter(yp, yp, 0, 64, dst_e2, dst_o2, dout // 16)

    wa, wb, b1, w2, b2 = _split_conv(p['conv1'])
    ab = _node_prep(x_p, batch3, mat_flat, wa, wb, b1)
    agg = edge_stage(ab, w2, b2)

    for conv_name, inter_name in (('conv2', 'inter1'),
                                  ('conv3', 'inter2'),
                                  ('conv4', 'inter3')):
        wa, wb, b1, w2, b2 = _split_conv(p[conv_name])
        it = p[inter_name]
        ab = _node_update(
            agg, it['l1']['w'], it['l1']['b'][None, :],
            it['l2']['w'], it['l2']['b'][None, :], wa, wb, b1)
        agg = edge_stage(ab, w2, b2)

    out128 = _final(agg, batch3, p['fc1']['w'], p['fc1']['b'][None, :])
    return out128[:, :1]


# speedup vs baseline: 1.4558x; 1.2662x over previous
import functools

import jax
import jax.numpy as jnp
from jax import lax
from jax.experimental import pallas as pl
from jax.experimental.pallas import tpu as pltpu
from jax.experimental.pallas import tpu_sc as plsc

F32 = jnp.float32
N_NODES = 50000
N_EDGES = 800000
N_GRAPHS = 64
NPAD = 50176
EPAD = 802816
NB = NPAD // 1024
EB = EPAD // 2048
NC, NS = 2, 16
NW = NC * NS
EPT = EPAD // NW
EPS = EPAD // NS
HPS = EPS // 2
RPT = NPAD // NS


def _lsilu(v, alpha):
    return v * jax.nn.sigmoid(v) + alpha * v


def _node_prep(x_p, batch3, mat_flat, wa, wb, b1):
    def body(x_ref, bt_ref, mat_ref, wa_ref, wb_ref, b1_ref, ab_ref):
        bt = bt_ref[0, 0, :]
        oh = (bt[:, None] == lax.broadcasted_iota(jnp.int32, (1024, N_GRAPHS), 1)
              ).astype(F32)
        M = jnp.dot(oh, mat_ref[...], preferred_element_type=F32)
        xb = x_ref[...]
        cols = [xb[:, 0:1]]
        for j in range(3):
            cols.append(xb[:, 1:2] * M[:, j:j + 1]
                        + xb[:, 2:3] * M[:, 3 + j:4 + j]
                        + xb[:, 3:4] * M[:, 6 + j:7 + j])
        h0 = jnp.concatenate(cols, axis=1)
        a = jnp.dot(h0, wa_ref[...], preferred_element_type=F32) + b1_ref[...]
        b = jnp.dot(h0, wb_ref[...], preferred_element_type=F32)
        ab_ref[...] = jnp.concatenate([a, b], axis=1)

    return pl.pallas_call(
        body,
        grid=(NB,),
        in_specs=[
            pl.BlockSpec((1024, 4), lambda i: (i, 0)),
            pl.BlockSpec((1, 1, 1024), lambda i: (i, 0, 0)),
            pl.BlockSpec((N_GRAPHS, 9), lambda i: (0, 0)),
            pl.BlockSpec((4, 64), lambda i: (0, 0)),
            pl.BlockSpec((4, 64), lambda i: (0, 0)),
            pl.BlockSpec((1, 64), lambda i: (0, 0)),
        ],
        out_specs=pl.BlockSpec((1024, 128), lambda i: (i, 0)),
        out_shape=jax.ShapeDtypeStruct((NPAD, 128), F32),
    )(x_p, batch3, mat_flat, wa, wb, b1)


def _node_update(agg, iw1, ib1, iw2, ib2, wa, wb, nb1):
    d = agg.shape[1]

    def body(g_ref, iw1_ref, ib1_ref, iw2_ref, ib2_ref, wa_ref, wb_ref,
             nb1_ref, ab_ref):
        h = _lsilu(g_ref[...], 0.1)
        hi = _lsilu(jnp.dot(h, iw1_ref[...], preferred_element_type=F32)
                    + ib1_ref[...], 0.05)
        hi = _lsilu(jnp.dot(hi, iw2_ref[...], preferred_element_type=F32)
                    + ib2_ref[...], 0.05)
        h = hi + h
        a = jnp.dot(h, wa_ref[...], preferred_element_type=F32) + nb1_ref[...]
        b = jnp.dot(h, wb_ref[...], preferred_element_type=F32)
        ab_ref[...] = jnp.concatenate([a, b], axis=1)

    return pl.pallas_call(
        body,
        grid=(NB,),
        in_specs=[
            pl.BlockSpec((1024, d), lambda i: (i, 0)),
            pl.BlockSpec(iw1.shape, lambda i: (0, 0)),
            pl.BlockSpec(ib1.shape, lambda i: (0, 0)),
            pl.BlockSpec(iw2.shape, lambda i: (0, 0)),
            pl.BlockSpec(ib2.shape, lambda i: (0, 0)),
            pl.BlockSpec((d, 64), lambda i: (0, 0)),
            pl.BlockSpec((d, 64), lambda i: (0, 0)),
            pl.BlockSpec((1, 64), lambda i: (0, 0)),
        ],
        out_specs=pl.BlockSpec((1024, 128), lambda i: (i, 0)),
        out_shape=jax.ShapeDtypeStruct((NPAD, 128), F32),
    )(agg, iw1, ib1, iw2, ib2, wa, wb, nb1)


def _edge_mm_packed(zp, w2, b2):
    dout = w2.shape[1]
    wbd = jnp.zeros((128, 128), F32)
    wbd = wbd.at[0:64, 0:dout].set(w2)
    wbd = wbd.at[64:128, 64:64 + dout].set(w2)
    bbd = jnp.zeros((1, 128), F32)
    bbd = bbd.at[:, 0:dout].set(b2)
    bbd = bbd.at[:, 64:64 + dout].set(b2)

    def body(z_ref, w_ref, b_ref, y_ref):
        za = _lsilu(z_ref[...], 0.05)
        yy = jnp.dot(za, w_ref[...], preferred_element_type=F32) + b_ref[...]
        y_ref[...] = _lsilu(yy, 0.05)

    return pl.pallas_call(
        body,
        grid=(EB // 2,),
        in_specs=[
            pl.BlockSpec((2048, 128), lambda i: (i, 0)),
            pl.BlockSpec((128, 128), lambda i: (0, 0)),
            pl.BlockSpec((1, 128), lambda i: (0, 0)),
        ],
        out_specs=pl.BlockSpec((2048, 128), lambda i: (i, 0)),
        out_shape=jax.ShapeDtypeStruct((EPAD // 2, 128), F32),
    )(zp, wbd, bbd)


def _edge_mm_split(zp, w2, b2):
    def body(z_ref, w_ref, b_ref, ye_ref, yo_ref):
        za = _lsilu(z_ref[...], 0.05)
        w = w_ref[...]
        b = b_ref[...]
        ye_ref[...] = _lsilu(
            jnp.dot(za[:, 0:64], w, preferred_element_type=F32) + b, 0.05)
        yo_ref[...] = _lsilu(
            jnp.dot(za[:, 64:128], w, preferred_element_type=F32) + b, 0.05)

    return pl.pallas_call(
        body,
        grid=(EB // 2,),
        in_specs=[
            pl.BlockSpec((2048, 128), lambda i: (i, 0)),
            pl.BlockSpec((64, 128), lambda i: (0, 0)),
            pl.BlockSpec((1, 128), lambda i: (0, 0)),
        ],
        out_specs=[pl.BlockSpec((2048, 128), lambda i: (i, 0))] * 2,
        out_shape=[jax.ShapeDtypeStruct((EPAD // 2, 128), F32)] * 2,
    )(zp, w2, b2)


def _final(agg, batch3, fcw, fcb):
    def body(g_ref, bt_ref, w_ref, b_ref, o_ref, acc_ref):
        i = pl.program_id(0)
        h = _lsilu(g_ref[...], 0.1)
        bt = bt_ref[0, 0, :]
        oh = (bt[:, None] == lax.broadcasted_iota(jnp.int32, (1024, N_GRAPHS), 1)
              ).astype(F32)
        part = lax.dot_general(oh, h, (((0,), (0,)), ((), ())),
                               preferred_element_type=F32)

        @pl.when(i == 0)
        def _():
            acc_ref[...] = part

        @pl.when(i > 0)
        def _():
            acc_ref[...] = acc_ref[...] + part

        @pl.when(i == NB - 1)
        def _():
            o = jnp.dot(acc_ref[...], w_ref[...], preferred_element_type=F32) \
                + b_ref[...]
            o = -_lsilu(o, 0.1) * 0.1
            o_ref[...] = jnp.broadcast_to(o, (N_GRAPHS, 128))

    return pl.pallas_call(
        body,
        grid=(NB,),
        in_specs=[
            pl.BlockSpec((1024, 128), lambda i: (i, 0)),
            pl.BlockSpec((1, 1, 1024), lambda i: (i, 0, 0)),
            pl.BlockSpec((128, 1), lambda i: (0, 0)),
            pl.BlockSpec((1, 1), lambda i: (0, 0)),
        ],
        out_specs=pl.BlockSpec((N_GRAPHS, 128), lambda i: (0, 0)),
        out_shape=jax.ShapeDtypeStruct((N_GRAPHS, 128), F32),
        scratch_shapes=[pltpu.VMEM((N_GRAPHS, 128), F32)],
    )(agg, batch3, fcw, fcb)


def _sc_gather(tab, dst_g, src_g):
    mesh = plsc.VectorSubcoreMesh(core_axis_name="c", subcore_axis_name="s")
    MC = 256
    NMC = EPT // MC
    PR = MC // 2

    @functools.partial(
        pl.kernel,
        out_type=jax.ShapeDtypeStruct((EPAD // 2, 128), F32),
        mesh=mesh,
        compiler_params=pltpu.CompilerParams(use_tc_tiling_on_sc=False),
        scratch_types=[
            pltpu.VMEM((2, 128), jnp.int32),
            pltpu.VMEM((2, 128), jnp.int32),
            pltpu.VMEM((2, 128), jnp.int32),
            pltpu.VMEM((2, 128), jnp.int32),
            pltpu.VMEM((MC, 64), F32),
            pltpu.VMEM((MC, 64), F32),
            pltpu.VMEM((MC, 64), F32),
            pltpu.VMEM((MC, 64), F32),
            pltpu.VMEM((PR, 128), F32),
            pltpu.VMEM((PR, 128), F32),
            pltpu.SemaphoreType.DMA,
            pltpu.SemaphoreType.DMA,
            pltpu.SemaphoreType.DMA,
            pltpu.SemaphoreType.DMA,
            pltpu.SemaphoreType.DMA,
            pltpu.SemaphoreType.DMA,
        ],
    )
    def k(t_hbm, d_hbm, s_hbm, z_hbm, idd0, idd1, ids0, ids1,
          ra0, ra1, rb0, rb1, zp0, zp1, six0, six1, sg0, sg1, st0, st1):
        idd = (idd0, idd1)
        ids_ = (ids0, ids1)
        ra = (ra0, ra1)
        rb = (rb0, rb1)
        zp = (zp0, zp1)
        six = (six0, six1)
        sg = (sg0, sg1)
        st = (st0, st1)
        wid = lax.axis_index("s") * NC + lax.axis_index("c")
        ebase = wid * EPT
        rbase = wid * (EPT // 128)

        def fire_idx(m, b):
            r0 = rbase + m * 2
            pltpu.async_copy(d_hbm.at[pl.ds(r0, 2)], idd[b], six[b])
            pltpu.async_copy(s_hbm.at[pl.ds(r0, 2)], ids_[b], six[b])

        def wait_idx(b):
            pltpu.make_async_copy(d_hbm.at[pl.ds(0, 2)], idd[b], six[b]).wait()
            pltpu.make_async_copy(s_hbm.at[pl.ds(0, 2)], ids_[b], six[b]).wait()

        def fire_gathers(b):
            for j in range(2):
                pltpu.async_copy(
                    t_hbm.at[idd[b].at[j]], ra[b].at[pl.ds(j * 128, 128)], sg[b])
                pltpu.async_copy(
                    t_hbm.at[ids_[b].at[j]], rb[b].at[pl.ds(j * 128, 128)], sg[b])

        def wait_gathers(b):
            pltpu.make_async_copy(t_hbm.at[pl.ds(0, MC)], ra[b], sg[b]).wait()
            pltpu.make_async_copy(t_hbm.at[pl.ds(0, MC)], rb[b], sg[b]).wait()

        def drain_store(b):
            pltpu.make_async_copy(zp[b], z_hbm.at[pl.ds(0, PR)], st[b]).wait()

        fire_idx(0, 0)
        fire_idx(1, 1)
        wait_idx(0)
        fire_gathers(0)

        def body(m2, carry):
            for b in range(2):
                m = 2 * m2 + b
                wait_gathers(b)

                @pl.when(m >= 2)
                def _(b=b):
                    drain_store(b)

                def addrow(q, c2, b=b):
                    for e in range(2):
                        for kk in range(4):
                            sl = pl.ds(kk * 16, 16)
                            zp[b][q, pl.ds(e * 64 + kk * 16, 16)] = (
                                ra[b][2 * q + e, sl] + rb[b][2 * q + e, sl])
                    return c2

                lax.fori_loop(0, PR, addrow, 0)
                off = ebase + m * MC
                pltpu.async_copy(zp[b], z_hbm.at[pl.ds(off // 2, PR)], st[b])
                bn = 1 - b

                @pl.when(m + 1 <= NMC - 1)
                def _(b=b, bn=bn):
                    wait_idx(bn)
                    fire_gathers(bn)

                @pl.when(m + 2 <= NMC - 1)
                def _(m=m, b=b):
                    fire_idx(m + 2, b)

            return carry

        lax.fori_loop(0, NMC // 2, body, 0)
        drain_store(0)
        drain_store(1)

    return k(tab, dst_g, src_g)


def _sc_scatter(ya, yb, cola, colb, dst_e2, dst_o2, nch):
    mesh = plsc.VectorSubcoreMesh(core_axis_name="c", subcore_axis_name="s")
    MC = 896
    NMC = HPS // MC
    NSUB = MC // 128
    IR = HPS // 128
    nhalf = nch // 2
    dreal = nch * 16

    @functools.partial(
        pl.kernel,
        out_type=jax.ShapeDtypeStruct((NPAD, dreal), F32),
        mesh=mesh,
        compiler_params=pltpu.CompilerParams(use_tc_tiling_on_sc=False),
        scratch_types=[
            pltpu.VMEM((NSUB, 128), jnp.int32),
            pltpu.VMEM((NSUB, 128), jnp.int32),
            pltpu.VMEM((MC, 16), F32),
            pltpu.VMEM((MC, 16), F32),
            pltpu.VMEM((128,), jnp.int32),
            pltpu.VMEM((128,), jnp.int32),
            pltpu.VMEM((128,), jnp.int32),
            pltpu.VMEM((128,), jnp.int32),
            pltpu.VMEM((128,), jnp.int32),
            pltpu.VMEM((128,), jnp.int32),
            pltpu.VMEM((128,), jnp.int32),
            pltpu.VMEM((392, 16), F32),
            pltpu.VMEM((392, 16), F32),
            pltpu.VMEM_SHARED((NPAD, 16), F32),
            pltpu.SemaphoreType.DMA,
            pltpu.SemaphoreType.DMA,
            pltpu.SemaphoreType.DMA,
            pltpu.SemaphoreType.DMA,
            pltpu.SemaphoreType.DMA,
        ],
    )
    def k(ya_hbm, yb_hbm, de_hbm, do_hbm, agg_hbm, idx0, idx1, yv0, yv1,
          q0, q1, q2, q3, q4, q5, q6, zb, bb, accum, sy0, sy1, ssc, six0,
          six1):
        yv = (yv0, yv1)
        sy = (sy0, sy1)
        six = (six0, six1)
        idxc = (idx0, idx1)
        idsb = (q0, q1, q2, q3, q4, q5, q6)
        core = lax.axis_index("c")
        sid = lax.axis_index("s")
        ebase = sid * HPS
        nbase = sid * RPT
        rbase = sid * IR

        def zrow(r, c2):
            zb[r, pl.ds(0, 16)] = jnp.zeros((16,), F32)
            return c2

        lax.fori_loop(0, 392, zrow, 0)

        for ci in range(nhalf):
            cc = 2 * ci + core
            for t in range(8):
                pltpu.sync_copy(zb, accum.at[pl.ds(nbase + t * 392, 392)])
            plsc.subcore_barrier()

            for y_hbm, d_hbm, colbase in ((ya_hbm, de_hbm, cola),
                                          (yb_hbm, do_hbm, colb)):
                col = colbase + cc * 16

                def fire_load(m, b, y_hbm=y_hbm, d_hbm=d_hbm, col=col):
                    off = ebase + m * MC
                    pltpu.async_copy(
                        y_hbm.at[pl.ds(off, MC), pl.ds(col, 16)], yv[b], sy[b])
                    pltpu.async_copy(
                        d_hbm.at[pl.ds(rbase + m * NSUB, NSUB)], idxc[b],
                        six[b])

                def wait_load(b, y_hbm=y_hbm, d_hbm=d_hbm, col=col):
                    pltpu.make_async_copy(
                        y_hbm.at[pl.ds(0, MC), pl.ds(col, 16)], yv[b],
                        sy[b]).wait()
                    pltpu.make_async_copy(
                        d_hbm.at[pl.ds(0, NSUB)], idxc[b], six[b]).wait()

                def drain_sc(b):
                    for q in range(NSUB):
                        pltpu.make_async_copy(
                            yv[b].at[pl.ds(q * 128, 128)],
                            accum.at[idsb[q]], ssc).wait()

                fire_load(0, 0)
                fire_load(1, 1)

                def body(m2, carry, fire_load=fire_load,
                         wait_load=wait_load, drain_sc=drain_sc):
                    for b in range(2):
                        m = 2 * m2 + b
                        wait_load(b)
                        for q in range(NSUB):
                            for kk in range(8):
                                sl = pl.ds(kk * 16, 16)
                                idsb[q][sl] = idxc[b][q, sl]
                            pltpu.async_copy(
                                yv[b].at[pl.ds(q * 128, 128)],
                                accum.at[idsb[q]], ssc, add=True)
                        drain_sc(b)

                        @pl.when(m + 2 <= NMC - 1)
                        def _(m=m, b=b):
                            fire_load(m + 2, b)

                    return carry

                lax.fori_loop(0, NMC // 2, body, 0)

            plsc.subcore_barrier()
            for t in range(8):
                r = nbase + t * 392
                pltpu.sync_copy(accum.at[pl.ds(r, 392)], bb)
                pltpu.sync_copy(
                    bb, agg_hbm.at[pl.ds(r, 392), pl.ds(cc * 16, 16)])
            plsc.subcore_barrier()

    return k(ya, yb, dst_e2, dst_o2)


def _split_conv(conv):
    w1 = conv['l1']['w']
    d = w1.shape[0] // 2
    return (w1[:d], w1[d:], conv['l1']['b'][None, :],
            conv['l2']['w'], conv['l2']['b'][None, :])


def kernel(x, matrix, batch, edge_index, params):
    p = params
    x_p = jnp.pad(x.astype(F32), ((0, NPAD - N_NODES), (0, 0)))
    batch_p = jnp.concatenate(
        [batch.astype(jnp.int32),
         jnp.full((NPAD - N_NODES,), N_GRAPHS, jnp.int32)])
    batch3 = batch_p.reshape(NB, 1, 1024)
    ei = edge_index.astype(jnp.int32)
    pad_e = jnp.full((EPAD - N_EDGES,), N_NODES, jnp.int32)
    src = jnp.concatenate([ei[0], pad_e])
    dst = jnp.concatenate([ei[1], pad_e])
    dst_g = (2 * dst).reshape(-1, 128)
    src_g = (2 * src + 1).reshape(-1, 128)
    dst_e2 = dst[0::2].reshape(-1, 128)
    dst_o2 = dst[1::2].reshape(-1, 128)
    mat_flat = matrix.astype(F32).reshape(N_GRAPHS, 9)

    def edge_stage(ab, w2, b2):
        tab = ab.reshape(2 * NPAD, 64)
        zp = _sc_gather(tab, dst_g, src_g)
        dout = w2.shape[1]
        if dout == 128:
            ye, yo = _edge_mm_split(zp, w2, b2)
            return _sc_scatter(ye, yo, 0, 0, dst_e2, dst_o2, dout // 16)
        yp = _edge_mm_packed(zp, w2, b2)
        return _sc_scatter(yp, yp, 0, 64, dst_e2, dst_o2, dout // 16)

    wa, wb, b1, w2, b2 = _split_conv(p['conv1'])
    ab = _node_prep(x_p, batch3, mat_flat, wa, wb, b1)
    agg = edge_stage(ab, w2, b2)

    for conv_name, inter_name in (('conv2', 'inter1'),
                                  ('conv3', 'inter2'),
                                  ('conv4', 'inter3')):
        wa, wb, b1, w2, b2 = _split_conv(p[conv_name])
        it = p[inter_name]
        ab = _node_update(
            agg, it['l1']['w'], it['l1']['b'][None, :],
            it['l2']['w'], it['l2']['b'][None, :], wa, wb, b1)
        agg = edge_stage(ab, w2, b2)

    out128 = _final(agg, batch3, p['fc1']['w'], p['fc1']['b'][None, :])
    return out128[:, :1]


# speedup vs baseline: 1.5389x; 1.0571x over previous
import functools

import jax
import jax.numpy as jnp
from jax import lax
from jax.experimental import pallas as pl
from jax.experimental.pallas import tpu as pltpu
from jax.experimental.pallas import tpu_sc as plsc

F32 = jnp.float32
N_NODES = 50000
N_EDGES = 800000
N_GRAPHS = 64
NPAD = 50176
EPAD = 802816
NB = NPAD // 1024
EB = EPAD // 2048
NC, NS = 2, 16
NW = NC * NS
EPT = EPAD // NW
EPS = EPAD // NS
HPS = EPS // 2
RPT = NPAD // NS


def _lsilu(v, alpha):
    return v * jax.nn.sigmoid(v) + alpha * v


def _node_prep(x_p, batch3, mat_flat, wa, wb, b1):
    def body(x_ref, bt_ref, mat_ref, wa_ref, wb_ref, b1_ref, ab_ref):
        bt = bt_ref[0, 0, :]
        oh = (bt[:, None] == lax.broadcasted_iota(jnp.int32, (1024, N_GRAPHS), 1)
              ).astype(F32)
        M = jnp.dot(oh, mat_ref[...], preferred_element_type=F32)
        xb = x_ref[...]
        cols = [xb[:, 0:1]]
        for j in range(3):
            cols.append(xb[:, 1:2] * M[:, j:j + 1]
                        + xb[:, 2:3] * M[:, 3 + j:4 + j]
                        + xb[:, 3:4] * M[:, 6 + j:7 + j])
        h0 = jnp.concatenate(cols, axis=1)
        a = jnp.dot(h0, wa_ref[...], preferred_element_type=F32) + b1_ref[...]
        b = jnp.dot(h0, wb_ref[...], preferred_element_type=F32)
        ab_ref[...] = jnp.concatenate([a, b], axis=1)

    return pl.pallas_call(
        body,
        grid=(NB,),
        in_specs=[
            pl.BlockSpec((1024, 4), lambda i: (i, 0)),
            pl.BlockSpec((1, 1, 1024), lambda i: (i, 0, 0)),
            pl.BlockSpec((N_GRAPHS, 9), lambda i: (0, 0)),
            pl.BlockSpec((4, 64), lambda i: (0, 0)),
            pl.BlockSpec((4, 64), lambda i: (0, 0)),
            pl.BlockSpec((1, 64), lambda i: (0, 0)),
        ],
        out_specs=pl.BlockSpec((1024, 128), lambda i: (i, 0)),
        out_shape=jax.ShapeDtypeStruct((NPAD, 128), F32),
    )(x_p, batch3, mat_flat, wa, wb, b1)


def _node_update(agg, iw1, ib1, iw2, ib2, wa, wb, nb1):
    d = agg.shape[1]

    def body(g_ref, iw1_ref, ib1_ref, iw2_ref, ib2_ref, wa_ref, wb_ref,
             nb1_ref, ab_ref):
        h = _lsilu(g_ref[...], 0.1)
        hi = _lsilu(jnp.dot(h, iw1_ref[...], preferred_element_type=F32)
                    + ib1_ref[...], 0.05)
        hi = _lsilu(jnp.dot(hi, iw2_ref[...], preferred_element_type=F32)
                    + ib2_ref[...], 0.05)
        h = hi + h
        a = jnp.dot(h, wa_ref[...], preferred_element_type=F32) + nb1_ref[...]
        b = jnp.dot(h, wb_ref[...], preferred_element_type=F32)
        ab_ref[...] = jnp.concatenate([a, b], axis=1)

    return pl.pallas_call(
        body,
        grid=(NB,),
        in_specs=[
            pl.BlockSpec((1024, d), lambda i: (i, 0)),
            pl.BlockSpec(iw1.shape, lambda i: (0, 0)),
            pl.BlockSpec(ib1.shape, lambda i: (0, 0)),
            pl.BlockSpec(iw2.shape, lambda i: (0, 0)),
            pl.BlockSpec(ib2.shape, lambda i: (0, 0)),
            pl.BlockSpec((d, 64), lambda i: (0, 0)),
            pl.BlockSpec((d, 64), lambda i: (0, 0)),
            pl.BlockSpec((1, 64), lambda i: (0, 0)),
        ],
        out_specs=pl.BlockSpec((1024, 128), lambda i: (i, 0)),
        out_shape=jax.ShapeDtypeStruct((NPAD, 128), F32),
    )(agg, iw1, ib1, iw2, ib2, wa, wb, nb1)


def _edge_mm_packed(zp, w2, b2):
    dout = w2.shape[1]
    wbd = jnp.zeros((128, 128), F32)
    wbd = wbd.at[0:64, 0:dout].set(w2)
    wbd = wbd.at[64:128, 64:64 + dout].set(w2)
    bbd = jnp.zeros((1, 128), F32)
    bbd = bbd.at[:, 0:dout].set(b2)
    bbd = bbd.at[:, 64:64 + dout].set(b2)

    def body(z_ref, w_ref, b_ref, y_ref):
        za = _lsilu(z_ref[...], 0.05)
        yy = jnp.dot(za, w_ref[...], preferred_element_type=F32) + b_ref[...]
        y_ref[...] = _lsilu(yy, 0.05)

    return pl.pallas_call(
        body,
        grid=(EB // 4,),
        in_specs=[
            pl.BlockSpec((4096, 128), lambda i: (i, 0)),
            pl.BlockSpec((128, 128), lambda i: (0, 0)),
            pl.BlockSpec((1, 128), lambda i: (0, 0)),
        ],
        out_specs=pl.BlockSpec((4096, 128), lambda i: (i, 0)),
        out_shape=jax.ShapeDtypeStruct((EPAD // 2, 128), F32),
    )(zp, wbd, bbd)


def _edge_mm_split(zp, w2, b2):
    def body(z_ref, w_ref, b_ref, ye_ref, yo_ref):
        za = _lsilu(z_ref[...], 0.05)
        w = w_ref[...]
        b = b_ref[...]
        ye_ref[...] = _lsilu(
            jnp.dot(za[:, 0:64], w, preferred_element_type=F32) + b, 0.05)
        yo_ref[...] = _lsilu(
            jnp.dot(za[:, 64:128], w, preferred_element_type=F32) + b, 0.05)

    return pl.pallas_call(
        body,
        grid=(EB // 4,),
        in_specs=[
            pl.BlockSpec((4096, 128), lambda i: (i, 0)),
            pl.BlockSpec((64, 128), lambda i: (0, 0)),
            pl.BlockSpec((1, 128), lambda i: (0, 0)),
        ],
        out_specs=[pl.BlockSpec((4096, 128), lambda i: (i, 0))] * 2,
        out_shape=[jax.ShapeDtypeStruct((EPAD // 2, 128), F32)] * 2,
    )(zp, w2, b2)


def _final(agg, batch3, fcw, fcb):
    def body(g_ref, bt_ref, w_ref, b_ref, o_ref, acc_ref):
        i = pl.program_id(0)
        h = _lsilu(g_ref[...], 0.1)
        bt = bt_ref[0, 0, :]
        oh = (bt[:, None] == lax.broadcasted_iota(jnp.int32, (1024, N_GRAPHS), 1)
              ).astype(F32)
        part = lax.dot_general(oh, h, (((0,), (0,)), ((), ())),
                               preferred_element_type=F32)

        @pl.when(i == 0)
        def _():
            acc_ref[...] = part

        @pl.when(i > 0)
        def _():
            acc_ref[...] = acc_ref[...] + part

        @pl.when(i == NB - 1)
        def _():
            o = jnp.dot(acc_ref[...], w_ref[...], preferred_element_type=F32) \
                + b_ref[...]
            o = -_lsilu(o, 0.1) * 0.1
            o_ref[...] = jnp.broadcast_to(o, (N_GRAPHS, 128))

    return pl.pallas_call(
        body,
        grid=(NB,),
        in_specs=[
            pl.BlockSpec((1024, 128), lambda i: (i, 0)),
            pl.BlockSpec((1, 1, 1024), lambda i: (i, 0, 0)),
            pl.BlockSpec((128, 1), lambda i: (0, 0)),
            pl.BlockSpec((1, 1), lambda i: (0, 0)),
        ],
        out_specs=pl.BlockSpec((N_GRAPHS, 128), lambda i: (0, 0)),
        out_shape=jax.ShapeDtypeStruct((N_GRAPHS, 128), F32),
        scratch_shapes=[pltpu.VMEM((N_GRAPHS, 128), F32)],
    )(agg, batch3, fcw, fcb)


def _sc_gather(tab, dst_g, src_g):
    mesh = plsc.VectorSubcoreMesh(core_axis_name="c", subcore_axis_name="s")
    MC = 256
    NMC = EPT // MC
    PR = MC // 2

    @functools.partial(
        pl.kernel,
        out_type=jax.ShapeDtypeStruct((EPAD // 2, 128), F32),
        mesh=mesh,
        compiler_params=pltpu.CompilerParams(use_tc_tiling_on_sc=False),
        scratch_types=[
            pltpu.VMEM((2, 128), jnp.int32),
            pltpu.VMEM((2, 128), jnp.int32),
            pltpu.VMEM((2, 128), jnp.int32),
            pltpu.VMEM((2, 128), jnp.int32),
            pltpu.VMEM((MC, 64), F32),
            pltpu.VMEM((MC, 64), F32),
            pltpu.VMEM((MC, 64), F32),
            pltpu.VMEM((MC, 64), F32),
            pltpu.VMEM((PR, 128), F32),
            pltpu.VMEM((PR, 128), F32),
            pltpu.SemaphoreType.DMA,
            pltpu.SemaphoreType.DMA,
            pltpu.SemaphoreType.DMA,
            pltpu.SemaphoreType.DMA,
            pltpu.SemaphoreType.DMA,
            pltpu.SemaphoreType.DMA,
        ],
    )
    def k(t_hbm, d_hbm, s_hbm, z_hbm, idd0, idd1, ids0, ids1,
          ra0, ra1, rb0, rb1, zp0, zp1, six0, six1, sg0, sg1, st0, st1):
        idd = (idd0, idd1)
        ids_ = (ids0, ids1)
        ra = (ra0, ra1)
        rb = (rb0, rb1)
        zp = (zp0, zp1)
        six = (six0, six1)
        sg = (sg0, sg1)
        st = (st0, st1)
        wid = lax.axis_index("s") * NC + lax.axis_index("c")
        ebase = wid * EPT
        rbase = wid * (EPT // 128)

        def fire_idx(m, b):
            r0 = rbase + m * 2
            pltpu.async_copy(d_hbm.at[pl.ds(r0, 2)], idd[b], six[b])
            pltpu.async_copy(s_hbm.at[pl.ds(r0, 2)], ids_[b], six[b])

        def wait_idx(b):
            pltpu.make_async_copy(d_hbm.at[pl.ds(0, 2)], idd[b], six[b]).wait()
            pltpu.make_async_copy(s_hbm.at[pl.ds(0, 2)], ids_[b], six[b]).wait()

        def fire_gathers(b):
            for j in range(2):
                pltpu.async_copy(
                    t_hbm.at[idd[b].at[j]], ra[b].at[pl.ds(j * 128, 128)], sg[b])
                pltpu.async_copy(
                    t_hbm.at[ids_[b].at[j]], rb[b].at[pl.ds(j * 128, 128)], sg[b])

        def wait_gathers(b):
            pltpu.make_async_copy(t_hbm.at[pl.ds(0, MC)], ra[b], sg[b]).wait()
            pltpu.make_async_copy(t_hbm.at[pl.ds(0, MC)], rb[b], sg[b]).wait()

        def drain_store(b):
            pltpu.make_async_copy(zp[b], z_hbm.at[pl.ds(0, PR)], st[b]).wait()

        fire_idx(0, 0)
        fire_idx(1, 1)
        wait_idx(0)
        fire_gathers(0)

        def body(m2, carry):
            for b in range(2):
                m = 2 * m2 + b
                wait_gathers(b)

                @pl.when(m >= 2)
                def _(b=b):
                    drain_store(b)

                def addrow(q, c2, b=b):
                    for e in range(2):
                        for kk in range(4):
                            sl = pl.ds(kk * 16, 16)
                            zp[b][q, pl.ds(e * 64 + kk * 16, 16)] = (
                                ra[b][2 * q + e, sl] + rb[b][2 * q + e, sl])
                    return c2

                lax.fori_loop(0, PR, addrow, 0)
                off = ebase + m * MC
                pltpu.async_copy(zp[b], z_hbm.at[pl.ds(off // 2, PR)], st[b])
                bn = 1 - b

                @pl.when(m + 1 <= NMC - 1)
                def _(b=b, bn=bn):
                    wait_idx(bn)
                    fire_gathers(bn)

                @pl.when(m + 2 <= NMC - 1)
                def _(m=m, b=b):
                    fire_idx(m + 2, b)

            return carry

        lax.fori_loop(0, NMC // 2, body, 0)
        drain_store(0)
        drain_store(1)

    return k(tab, dst_g, src_g)


def _sc_scatter(ya, yb, cola, colb, dst_e2, dst_o2, nch):
    mesh = plsc.VectorSubcoreMesh(core_axis_name="c", subcore_axis_name="s")
    MC = 896
    NMC = HPS // MC
    NSUB = MC // 128
    IR = HPS // 128
    nhalf = nch // 2
    dreal = nch * 16

    @functools.partial(
        pl.kernel,
        out_type=jax.ShapeDtypeStruct((NPAD, dreal), F32),
        mesh=mesh,
        compiler_params=pltpu.CompilerParams(use_tc_tiling_on_sc=False),
        scratch_types=[
            pltpu.VMEM((NSUB, 128), jnp.int32),
            pltpu.VMEM((NSUB, 128), jnp.int32),
            pltpu.VMEM((MC, 16), F32),
            pltpu.VMEM((MC, 16), F32),
            pltpu.VMEM((128,), jnp.int32),
            pltpu.VMEM((128,), jnp.int32),
            pltpu.VMEM((128,), jnp.int32),
            pltpu.VMEM((128,), jnp.int32),
            pltpu.VMEM((128,), jnp.int32),
            pltpu.VMEM((128,), jnp.int32),
            pltpu.VMEM((128,), jnp.int32),
            pltpu.VMEM((392, 16), F32),
            pltpu.VMEM((392, 16), F32),
            pltpu.VMEM_SHARED((NPAD, 16), F32),
            pltpu.SemaphoreType.DMA,
            pltpu.SemaphoreType.DMA,
            pltpu.SemaphoreType.DMA,
            pltpu.SemaphoreType.DMA,
            pltpu.SemaphoreType.DMA,
        ],
    )
    def k(ya_hbm, yb_hbm, de_hbm, do_hbm, agg_hbm, idx0, idx1, yv0, yv1,
          q0, q1, q2, q3, q4, q5, q6, zb, bb, accum, sy0, sy1, ssc, six0,
          six1):
        yv = (yv0, yv1)
        sy = (sy0, sy1)
        six = (six0, six1)
        idxc = (idx0, idx1)
        idsb = (q0, q1, q2, q3, q4, q5, q6)
        core = lax.axis_index("c")
        sid = lax.axis_index("s")
        ebase = sid * HPS
        nbase = sid * RPT
        rbase = sid * IR

        def zrow(r, c2):
            zb[r, pl.ds(0, 16)] = jnp.zeros((16,), F32)
            return c2

        lax.fori_loop(0, 392, zrow, 0)

        for ci in range(nhalf):
            cc = 2 * ci + core
            for t in range(8):
                pltpu.sync_copy(zb, accum.at[pl.ds(nbase + t * 392, 392)])
            plsc.subcore_barrier()

            for y_hbm, d_hbm, colbase in ((ya_hbm, de_hbm, cola),
                                          (yb_hbm, do_hbm, colb)):
                col = colbase + cc * 16

                def fire_load(m, b, y_hbm=y_hbm, d_hbm=d_hbm, col=col):
                    off = ebase + m * MC
                    pltpu.async_copy(
                        y_hbm.at[pl.ds(off, MC), pl.ds(col, 16)], yv[b], sy[b])
                    pltpu.async_copy(
                        d_hbm.at[pl.ds(rbase + m * NSUB, NSUB)], idxc[b],
                        six[b])

                def wait_load(b, y_hbm=y_hbm, d_hbm=d_hbm, col=col):
                    pltpu.make_async_copy(
                        y_hbm.at[pl.ds(0, MC), pl.ds(col, 16)], yv[b],
                        sy[b]).wait()
                    pltpu.make_async_copy(
                        d_hbm.at[pl.ds(0, NSUB)], idxc[b], six[b]).wait()

                def drain_sc(b):
                    for q in range(NSUB):
                        pltpu.make_async_copy(
                            yv[b].at[pl.ds(q * 128, 128)],
                            accum.at[idsb[q]], ssc).wait()

                fire_load(0, 0)
                fire_load(1, 1)

                def body(m2, carry, fire_load=fire_load,
                         wait_load=wait_load, drain_sc=drain_sc):
                    for b in range(2):
                        m = 2 * m2 + b
                        wait_load(b)
                        for q in range(NSUB):
                            for kk in range(8):
                                sl = pl.ds(kk * 16, 16)
                                idsb[q][sl] = idxc[b][q, sl]
                            pltpu.async_copy(
                                yv[b].at[pl.ds(q * 128, 128)],
                                accum.at[idsb[q]], ssc, add=True)
                        drain_sc(b)

                        @pl.when(m + 2 <= NMC - 1)
                        def _(m=m, b=b):
                            fire_load(m + 2, b)

                    return carry

                lax.fori_loop(0, NMC // 2, body, 0)

            plsc.subcore_barrier()
            for t in range(8):
                r = nbase + t * 392
                pltpu.sync_copy(accum.at[pl.ds(r, 392)], bb)
                pltpu.sync_copy(
                    bb, agg_hbm.at[pl.ds(r, 392), pl.ds(cc * 16, 16)])
            plsc.subcore_barrier()

    return k(ya, yb, dst_e2, dst_o2)


def _split_conv(conv):
    w1 = conv['l1']['w']
    d = w1.shape[0] // 2
    return (w1[:d], w1[d:], conv['l1']['b'][None, :],
            conv['l2']['w'], conv['l2']['b'][None, :])


def kernel(x, matrix, batch, edge_index, params):
    p = params
    x_p = jnp.pad(x.astype(F32), ((0, NPAD - N_NODES), (0, 0)))
    batch_p = jnp.concatenate(
        [batch.astype(jnp.int32),
         jnp.full((NPAD - N_NODES,), N_GRAPHS, jnp.int32)])
    batch3 = batch_p.reshape(NB, 1, 1024)
    ei = edge_index.astype(jnp.int32)
    pad_e = jnp.full((EPAD - N_EDGES,), N_NODES, jnp.int32)
    src = jnp.concatenate([ei[0], pad_e])
    dst = jnp.concatenate([ei[1], pad_e])
    dst_g = (2 * dst).reshape(-1, 128)
    src_g = (2 * src + 1).reshape(-1, 128)
    dst_e2 = dst[0::2].reshape(-1, 128)
    dst_o2 = dst[1::2].reshape(-1, 128)
    mat_flat = matrix.astype(F32).reshape(N_GRAPHS, 9)

    def edge_stage(ab, w2, b2):
        tab = ab.reshape(2 * NPAD, 64)
        zp = _sc_gather(tab, dst_g, src_g)
        dout = w2.shape[1]
        if dout == 128:
            ye, yo = _edge_mm_split(zp, w2, b2)
            return _sc_scatter(ye, yo, 0, 0, dst_e2, dst_o2, dout // 16)
        yp = _edge_mm_packed(zp, w2, b2)
        return _sc_scatter(yp, yp, 0, 64, dst_e2, dst_o2, dout // 16)

    wa, wb, b1, w2, b2 = _split_conv(p['conv1'])
    ab = _node_prep(x_p, batch3, mat_flat, wa, wb, b1)
    agg = edge_stage(ab, w2, b2)

    for conv_name, inter_name in (('conv2', 'inter1'),
                                  ('conv3', 'inter2'),
                                  ('conv4', 'inter3')):
        wa, wb, b1, w2, b2 = _split_conv(p[conv_name])
        it = p[inter_name]
        ab = _node_update(
            agg, it['l1']['w'], it['l1']['b'][None, :],
            it['l2']['w'], it['l2']['b'][None, :], wa, wb, b1)
        agg = edge_stage(ab, w2, b2)

    out128 = _final(agg, batch3, p['fc1']['w'], p['fc1']['b'][None, :])
    return out128[:, :1]


# speedup vs baseline: 1.5737x; 1.0226x over previous
import functools

import jax
import jax.numpy as jnp
from jax import lax
from jax.experimental import pallas as pl
from jax.experimental.pallas import tpu as pltpu
from jax.experimental.pallas import tpu_sc as plsc

F32 = jnp.float32
N_NODES = 50000
N_EDGES = 800000
N_GRAPHS = 64
NPAD = 50176
EPAD = 802816
NB = NPAD // 1024
EB = EPAD // 2048
NC, NS = 2, 16
NW = NC * NS
EPT = EPAD // NW
EPS = EPAD // NS
HPS = EPS // 2
RPT = NPAD // NS


def _lsilu(v, alpha):
    return v * jax.nn.sigmoid(v) + alpha * v


def _node_prep(x_p, batch3, mat_flat, wa, wb, b1):
    def body(x_ref, bt_ref, mat_ref, wa_ref, wb_ref, b1_ref, ab_ref):
        bt = bt_ref[0, 0, :]
        oh = (bt[:, None] == lax.broadcasted_iota(jnp.int32, (1024, N_GRAPHS), 1)
              ).astype(F32)
        M = jnp.dot(oh, mat_ref[...], preferred_element_type=F32)
        xb = x_ref[...]
        cols = [xb[:, 0:1]]
        for j in range(3):
            cols.append(xb[:, 1:2] * M[:, j:j + 1]
                        + xb[:, 2:3] * M[:, 3 + j:4 + j]
                        + xb[:, 3:4] * M[:, 6 + j:7 + j])
        h0 = jnp.concatenate(cols, axis=1)
        a = jnp.dot(h0, wa_ref[...], preferred_element_type=F32) + b1_ref[...]
        b = jnp.dot(h0, wb_ref[...], preferred_element_type=F32)
        ab_ref[...] = jnp.concatenate([a, b], axis=1)

    return pl.pallas_call(
        body,
        grid=(NB,),
        in_specs=[
            pl.BlockSpec((1024, 4), lambda i: (i, 0)),
            pl.BlockSpec((1, 1, 1024), lambda i: (i, 0, 0)),
            pl.BlockSpec((N_GRAPHS, 9), lambda i: (0, 0)),
            pl.BlockSpec((4, 64), lambda i: (0, 0)),
            pl.BlockSpec((4, 64), lambda i: (0, 0)),
            pl.BlockSpec((1, 64), lambda i: (0, 0)),
        ],
        out_specs=pl.BlockSpec((1024, 128), lambda i: (i, 0)),
        out_shape=jax.ShapeDtypeStruct((NPAD, 128), F32),
    )(x_p, batch3, mat_flat, wa, wb, b1)


def _node_update(agg, iw1, ib1, iw2, ib2, wa, wb, nb1):
    d = agg.shape[1]

    def body(g_ref, iw1_ref, ib1_ref, iw2_ref, ib2_ref, wa_ref, wb_ref,
             nb1_ref, ab_ref):
        h = _lsilu(g_ref[...], 0.1)
        hi = _lsilu(jnp.dot(h, iw1_ref[...], preferred_element_type=F32)
                    + ib1_ref[...], 0.05)
        hi = _lsilu(jnp.dot(hi, iw2_ref[...], preferred_element_type=F32)
                    + ib2_ref[...], 0.05)
        h = hi + h
        a = jnp.dot(h, wa_ref[...], preferred_element_type=F32) + nb1_ref[...]
        b = jnp.dot(h, wb_ref[...], preferred_element_type=F32)
        ab_ref[...] = jnp.concatenate([a, b], axis=1)

    return pl.pallas_call(
        body,
        grid=(NB,),
        in_specs=[
            pl.BlockSpec((1024, d), lambda i: (i, 0)),
            pl.BlockSpec(iw1.shape, lambda i: (0, 0)),
            pl.BlockSpec(ib1.shape, lambda i: (0, 0)),
            pl.BlockSpec(iw2.shape, lambda i: (0, 0)),
            pl.BlockSpec(ib2.shape, lambda i: (0, 0)),
            pl.BlockSpec((d, 64), lambda i: (0, 0)),
            pl.BlockSpec((d, 64), lambda i: (0, 0)),
            pl.BlockSpec((1, 64), lambda i: (0, 0)),
        ],
        out_specs=pl.BlockSpec((1024, 128), lambda i: (i, 0)),
        out_shape=jax.ShapeDtypeStruct((NPAD, 128), F32),
    )(agg, iw1, ib1, iw2, ib2, wa, wb, nb1)


def _edge_mm_packed(zp, w2, b2):
    dout = w2.shape[1]
    wbd = jnp.zeros((128, 128), F32)
    wbd = wbd.at[0:64, 0:dout].set(w2)
    wbd = wbd.at[64:128, 64:64 + dout].set(w2)
    bbd = jnp.zeros((1, 128), F32)
    bbd = bbd.at[:, 0:dout].set(b2)
    bbd = bbd.at[:, 64:64 + dout].set(b2)

    def body(z_ref, w_ref, b_ref, y_ref):
        za = _lsilu(z_ref[...], 0.05)
        yy = jnp.dot(za, w_ref[...], preferred_element_type=F32) + b_ref[...]
        y_ref[...] = _lsilu(yy, 0.05)

    return pl.pallas_call(
        body,
        grid=(EB // 8,),
        in_specs=[
            pl.BlockSpec((8192, 128), lambda i: (i, 0)),
            pl.BlockSpec((128, 128), lambda i: (0, 0)),
            pl.BlockSpec((1, 128), lambda i: (0, 0)),
        ],
        out_specs=pl.BlockSpec((8192, 128), lambda i: (i, 0)),
        out_shape=jax.ShapeDtypeStruct((EPAD // 2, 128), F32),
    )(zp, wbd, bbd)


def _edge_mm_split(zp, w2, b2):
    def body(z_ref, w_ref, b_ref, ye_ref, yo_ref):
        za = _lsilu(z_ref[...], 0.05)
        w = w_ref[...]
        b = b_ref[...]
        ye_ref[...] = _lsilu(
            jnp.dot(za[:, 0:64], w, preferred_element_type=F32) + b, 0.05)
        yo_ref[...] = _lsilu(
            jnp.dot(za[:, 64:128], w, preferred_element_type=F32) + b, 0.05)

    return pl.pallas_call(
        body,
        grid=(EB // 8,),
        in_specs=[
            pl.BlockSpec((8192, 128), lambda i: (i, 0)),
            pl.BlockSpec((64, 128), lambda i: (0, 0)),
            pl.BlockSpec((1, 128), lambda i: (0, 0)),
        ],
        out_specs=[pl.BlockSpec((8192, 128), lambda i: (i, 0))] * 2,
        out_shape=[jax.ShapeDtypeStruct((EPAD // 2, 128), F32)] * 2,
    )(zp, w2, b2)


def _final(agg, batch3, fcw, fcb):
    def body(g_ref, bt_ref, w_ref, b_ref, o_ref, acc_ref):
        i = pl.program_id(0)
        h = _lsilu(g_ref[...], 0.1)
        bt = bt_ref[0, 0, :]
        oh = (bt[:, None] == lax.broadcasted_iota(jnp.int32, (1024, N_GRAPHS), 1)
              ).astype(F32)
        part = lax.dot_general(oh, h, (((0,), (0,)), ((), ())),
                               preferred_element_type=F32)

        @pl.when(i == 0)
        def _():
            acc_ref[...] = part

        @pl.when(i > 0)
        def _():
            acc_ref[...] = acc_ref[...] + part

        @pl.when(i == NB - 1)
        def _():
            o = jnp.dot(acc_ref[...], w_ref[...], preferred_element_type=F32) \
                + b_ref[...]
            o = -_lsilu(o, 0.1) * 0.1
            o_ref[...] = jnp.broadcast_to(o, (N_GRAPHS, 128))

    return pl.pallas_call(
        body,
        grid=(NB,),
        in_specs=[
            pl.BlockSpec((1024, 128), lambda i: (i, 0)),
            pl.BlockSpec((1, 1, 1024), lambda i: (i, 0, 0)),
            pl.BlockSpec((128, 1), lambda i: (0, 0)),
            pl.BlockSpec((1, 1), lambda i: (0, 0)),
        ],
        out_specs=pl.BlockSpec((N_GRAPHS, 128), lambda i: (0, 0)),
        out_shape=jax.ShapeDtypeStruct((N_GRAPHS, 128), F32),
        scratch_shapes=[pltpu.VMEM((N_GRAPHS, 128), F32)],
    )(agg, batch3, fcw, fcb)


def _sc_gather(tab, dst_g, src_g):
    mesh = plsc.VectorSubcoreMesh(core_axis_name="c", subcore_axis_name="s")
    MC = 256
    NMC = EPT // MC
    PR = MC // 2

    @functools.partial(
        pl.kernel,
        out_type=jax.ShapeDtypeStruct((EPAD // 2, 128), F32),
        mesh=mesh,
        compiler_params=pltpu.CompilerParams(use_tc_tiling_on_sc=False),
        scratch_types=[
            pltpu.VMEM((2, 128), jnp.int32),
            pltpu.VMEM((2, 128), jnp.int32),
            pltpu.VMEM((2, 128), jnp.int32),
            pltpu.VMEM((2, 128), jnp.int32),
            pltpu.VMEM((MC, 64), F32),
            pltpu.VMEM((MC, 64), F32),
            pltpu.VMEM((MC, 64), F32),
            pltpu.VMEM((MC, 64), F32),
            pltpu.VMEM((PR, 128), F32),
            pltpu.VMEM((PR, 128), F32),
            pltpu.SemaphoreType.DMA,
            pltpu.SemaphoreType.DMA,
            pltpu.SemaphoreType.DMA,
            pltpu.SemaphoreType.DMA,
            pltpu.SemaphoreType.DMA,
            pltpu.SemaphoreType.DMA,
        ],
    )
    def k(t_hbm, d_hbm, s_hbm, z_hbm, idd0, idd1, ids0, ids1,
          ra0, ra1, rb0, rb1, zp0, zp1, six0, six1, sg0, sg1, st0, st1):
        idd = (idd0, idd1)
        ids_ = (ids0, ids1)
        ra = (ra0, ra1)
        rb = (rb0, rb1)
        zp = (zp0, zp1)
        six = (six0, six1)
        sg = (sg0, sg1)
        st = (st0, st1)
        wid = lax.axis_index("s") * NC + lax.axis_index("c")
        ebase = wid * EPT
        rbase = wid * (EPT // 128)

        def fire_idx(m, b):
            r0 = rbase + m * 2
            pltpu.async_copy(d_hbm.at[pl.ds(r0, 2)], idd[b], six[b])
            pltpu.async_copy(s_hbm.at[pl.ds(r0, 2)], ids_[b], six[b])

        def wait_idx(b):
            pltpu.make_async_copy(d_hbm.at[pl.ds(0, 2)], idd[b], six[b]).wait()
            pltpu.make_async_copy(s_hbm.at[pl.ds(0, 2)], ids_[b], six[b]).wait()

        def fire_gathers(b):
            for j in range(2):
                pltpu.async_copy(
                    t_hbm.at[idd[b].at[j]], ra[b].at[pl.ds(j * 128, 128)], sg[b])
                pltpu.async_copy(
                    t_hbm.at[ids_[b].at[j]], rb[b].at[pl.ds(j * 128, 128)], sg[b])

        def wait_gathers(b):
            pltpu.make_async_copy(t_hbm.at[pl.ds(0, MC)], ra[b], sg[b]).wait()
            pltpu.make_async_copy(t_hbm.at[pl.ds(0, MC)], rb[b], sg[b]).wait()

        def drain_store(b):
            pltpu.make_async_copy(zp[b], z_hbm.at[pl.ds(0, PR)], st[b]).wait()

        fire_idx(0, 0)
        fire_idx(1, 1)
        wait_idx(0)
        fire_gathers(0)

        def body(m2, carry):
            for b in range(2):
                m = 2 * m2 + b
                wait_gathers(b)

                @pl.when(m >= 2)
                def _(b=b):
                    drain_store(b)

                def addrow(q, c2, b=b):
                    for e in range(2):
                        for kk in range(4):
                            sl = pl.ds(kk * 16, 16)
                            zp[b][q, pl.ds(e * 64 + kk * 16, 16)] = (
                                ra[b][2 * q + e, sl] + rb[b][2 * q + e, sl])
                    return c2

                lax.fori_loop(0, PR, addrow, 0)
                off = ebase + m * MC
                pltpu.async_copy(zp[b], z_hbm.at[pl.ds(off // 2, PR)], st[b])
                bn = 1 - b

                @pl.when(m + 1 <= NMC - 1)
                def _(b=b, bn=bn):
                    wait_idx(bn)
                    fire_gathers(bn)

                @pl.when(m + 2 <= NMC - 1)
                def _(m=m, b=b):
                    fire_idx(m + 2, b)

            return carry

        lax.fori_loop(0, NMC // 2, body, 0)
        drain_store(0)
        drain_store(1)

    return k(tab, dst_g, src_g)


def _sc_scatter(ya, yb, cola, colb, dst_e2, dst_o2, nch):
    mesh = plsc.VectorSubcoreMesh(core_axis_name="c", subcore_axis_name="s")
    MC = 896
    NMC = HPS // MC
    NSUB = MC // 128
    IR = HPS // 128
    nhalf = nch // 2
    dreal = nch * 16

    @functools.partial(
        pl.kernel,
        out_type=jax.ShapeDtypeStruct((NPAD, dreal), F32),
        mesh=mesh,
        compiler_params=pltpu.CompilerParams(use_tc_tiling_on_sc=False),
        scratch_types=[
            pltpu.VMEM((NSUB, 128), jnp.int32),
            pltpu.VMEM((NSUB, 128), jnp.int32),
            pltpu.VMEM((MC, 16), F32),
            pltpu.VMEM((MC, 16), F32),
            pltpu.VMEM((128,), jnp.int32),
            pltpu.VMEM((128,), jnp.int32),
            pltpu.VMEM((128,), jnp.int32),
            pltpu.VMEM((128,), jnp.int32),
            pltpu.VMEM((128,), jnp.int32),
            pltpu.VMEM((128,), jnp.int32),
            pltpu.VMEM((128,), jnp.int32),
            pltpu.VMEM((392, 16), F32),
            pltpu.VMEM((392, 16), F32),
            pltpu.VMEM_SHARED((NPAD, 16), F32),
            pltpu.SemaphoreType.DMA,
            pltpu.SemaphoreType.DMA,
            pltpu.SemaphoreType.DMA,
            pltpu.SemaphoreType.DMA,
            pltpu.SemaphoreType.DMA,
        ],
    )
    def k(ya_hbm, yb_hbm, de_hbm, do_hbm, agg_hbm, idx0, idx1, yv0, yv1,
          q0, q1, q2, q3, q4, q5, q6, zb, bb, accum, sy0, sy1, ssc, six0,
          six1):
        yv = (yv0, yv1)
        sy = (sy0, sy1)
        six = (six0, six1)
        idxc = (idx0, idx1)
        idsb = (q0, q1, q2, q3, q4, q5, q6)
        core = lax.axis_index("c")
        sid = lax.axis_index("s")
        ebase = sid * HPS
        nbase = sid * RPT
        rbase = sid * IR

        def zrow(r, c2):
            zb[r, pl.ds(0, 16)] = jnp.zeros((16,), F32)
            return c2

        lax.fori_loop(0, 392, zrow, 0)

        for ci in range(nhalf):
            cc = 2 * ci + core
            for t in range(8):
                pltpu.sync_copy(zb, accum.at[pl.ds(nbase + t * 392, 392)])
            plsc.subcore_barrier()

            for y_hbm, d_hbm, colbase in ((ya_hbm, de_hbm, cola),
                                          (yb_hbm, do_hbm, colb)):
                col = colbase + cc * 16

                def fire_load(m, b, y_hbm=y_hbm, d_hbm=d_hbm, col=col):
                    off = ebase + m * MC
                    pltpu.async_copy(
                        y_hbm.at[pl.ds(off, MC), pl.ds(col, 16)], yv[b], sy[b])
                    pltpu.async_copy(
                        d_hbm.at[pl.ds(rbase + m * NSUB, NSUB)], idxc[b],
                        six[b])

                def wait_load(b, y_hbm=y_hbm, d_hbm=d_hbm, col=col):
                    pltpu.make_async_copy(
                        y_hbm.at[pl.ds(0, MC), pl.ds(col, 16)], yv[b],
                        sy[b]).wait()
                    pltpu.make_async_copy(
                        d_hbm.at[pl.ds(0, NSUB)], idxc[b], six[b]).wait()

                def drain_sc(b):
                    for q in range(NSUB):
                        pltpu.make_async_copy(
                            yv[b].at[pl.ds(q * 128, 128)],
                            accum.at[idsb[q]], ssc).wait()

                fire_load(0, 0)
                fire_load(1, 1)

                def body(m2, carry, fire_load=fire_load,
                         wait_load=wait_load, drain_sc=drain_sc):
                    for b in range(2):
                        m = 2 * m2 + b
                        wait_load(b)
                        for q in range(NSUB):
                            for kk in range(8):
                                sl = pl.ds(kk * 16, 16)
                                idsb[q][sl] = idxc[b][q, sl]
                            pltpu.async_copy(
                                yv[b].at[pl.ds(q * 128, 128)],
                                accum.at[idsb[q]], ssc, add=True)
                        drain_sc(b)

                        @pl.when(m + 2 <= NMC - 1)
                        def _(m=m, b=b):
                            fire_load(m + 2, b)

                    return carry

                lax.fori_loop(0, NMC // 2, body, 0)

            plsc.subcore_barrier()
            for t in range(8):
                r = nbase + t * 392
                pltpu.sync_copy(accum.at[pl.ds(r, 392)], bb)
                pltpu.sync_copy(
                    bb, agg_hbm.at[pl.ds(r, 392), pl.ds(cc * 16, 16)])
            plsc.subcore_barrier()

    return k(ya, yb, dst_e2, dst_o2)


def _split_conv(conv):
    w1 = conv['l1']['w']
    d = w1.shape[0] // 2
    return (w1[:d], w1[d:], conv['l1']['b'][None, :],
            conv['l2']['w'], conv['l2']['b'][None, :])


def kernel(x, matrix, batch, edge_index, params):
    p = params
    x_p = jnp.pad(x.astype(F32), ((0, NPAD - N_NODES), (0, 0)))
    batch_p = jnp.concatenate(
        [batch.astype(jnp.int32),
         jnp.full((NPAD - N_NODES,), N_GRAPHS, jnp.int32)])
    batch3 = batch_p.reshape(NB, 1, 1024)
    ei = edge_index.astype(jnp.int32)
    pad_e = jnp.full((EPAD - N_EDGES,), N_NODES, jnp.int32)
    src = jnp.concatenate([ei[0], pad_e])
    dst = jnp.concatenate([ei[1], pad_e])
    dst_g = (2 * dst).reshape(-1, 128)
    src_g = (2 * src + 1).reshape(-1, 128)
    dst_e2 = dst[0::2].reshape(-1, 128)
    dst_o2 = dst[1::2].reshape(-1, 128)
    mat_flat = matrix.astype(F32).reshape(N_GRAPHS, 9)

    def edge_stage(ab, w2, b2):
        tab = ab.reshape(2 * NPAD, 64)
        zp = _sc_gather(tab, dst_g, src_g)
        dout = w2.shape[1]
        if dout == 128:
            ye, yo = _edge_mm_split(zp, w2, b2)
            return _sc_scatter(ye, yo, 0, 0, dst_e2, dst_o2, dout // 16)
        yp = _edge_mm_packed(zp, w2, b2)
        return _sc_scatter(yp, yp, 0, 64, dst_e2, dst_o2, dout // 16)

    wa, wb, b1, w2, b2 = _split_conv(p['conv1'])
    ab = _node_prep(x_p, batch3, mat_flat, wa, wb, b1)
    agg = edge_stage(ab, w2, b2)

    for conv_name, inter_name in (('conv2', 'inter1'),
                                  ('conv3', 'inter2'),
                                  ('conv4', 'inter3')):
        wa, wb, b1, w2, b2 = _split_conv(p[conv_name])
        it = p[inter_name]
        ab = _node_update(
            agg, it['l1']['w'], it['l1']['b'][None, :],
            it['l2']['w'], it['l2']['b'][None, :], wa, wb, b1)
        agg = edge_stage(ab, w2, b2)

    out128 = _final(agg, batch3, p['fc1']['w'], p['fc1']['b'][None, :])
    return out128[:, :1]
